# Initial kernel scaffold; baseline (speedup 1.0000x reference)
import functools

import jax
import jax.numpy as jnp
from jax import lax
from jax.experimental import pallas as pl
from jax.experimental.pallas import tpu as pltpu
from jax.experimental.pallas import tpu_sc as plsc

N_NODES = 10000
NP = 10240
D = 16
F = 128
E = 320000
NW = 32
EW = 10240
EP = NW * EW
CH = 128
NCH = EW // CH
NF = NP // D

_mesh = plsc.VectorSubcoreMesh(core_axis_name="c", subcore_axis_name="s")


@functools.partial(
    pl.kernel,
    out_type=jax.ShapeDtypeStruct((2, 2, NF, D), jnp.float32),
    mesh=_mesh,
    scratch_types=[
        pltpu.VMEM((EW,), jnp.int32),
        pltpu.VMEM((NF, D), jnp.float32),
        pltpu.VMEM((NF // CH, CH), jnp.int32),
        pltpu.VMEM_SHARED((NF, D), jnp.float32),
        pltpu.VMEM_SHARED((NF, D), jnp.float32),
    ],
)
def _deg_kernel(dp_hbm, dn_hbm, z_hbm, out_hbm, idx_v, hist_v, rix_v, accp, accn):
    cid = lax.axis_index("c")
    sid = lax.axis_index("s")
    wid = cid * 16 + sid
    iota16 = lax.iota(jnp.int32, 16)
    ones16 = jnp.ones((16,), jnp.float32)
    for c in range(NF // CH):
        for k in range(CH // 16):
            rix_v[c, pl.ds(k * 16, 16)] = iota16 + (c * CH + k * 16)
    st = NF // 16
    pltpu.sync_copy(z_hbm.at[pl.ds(sid * st, st)], accp.at[pl.ds(sid * st, st)])
    pltpu.sync_copy(z_hbm.at[pl.ds(sid * st, st)], accn.at[pl.ds(sid * st, st)])
    plsc.subcore_barrier()

    def one_sign(d_hbm, acc):
        pltpu.sync_copy(z_hbm.at[pl.ds(0, NF)], hist_v)
        pltpu.sync_copy(d_hbm.at[wid], idx_v)

        def body(e, carry):
            iv = idx_v[pl.ds(e * 16, 16)]
            row = jnp.right_shift(iv, 4)
            col = jnp.bitwise_and(iv, 15)
            plsc.addupdate_scatter(hist_v, [row, col], ones16)
            return carry

        lax.fori_loop(0, EW // 16, body, 0)
        for c in range(NF // CH):
            pltpu.sync_copy(hist_v.at[pl.ds(c * CH, CH)],
                            acc.at[rix_v.at[c]], add=True)

    one_sign(dp_hbm, accp)
    one_sign(dn_hbm, accn)
    plsc.subcore_barrier()
    pltpu.sync_copy(accp.at[pl.ds(sid * st, st)],
                    out_hbm.at[cid, 0, pl.ds(sid * st, st)])
    pltpu.sync_copy(accn.at[pl.ds(sid * st, st)],
                    out_hbm.at[cid, 1, pl.ds(sid * st, st)])


@functools.partial(
    pl.kernel,
    out_type=[jax.ShapeDtypeStruct((2, NP, D), jnp.float32),
              jax.ShapeDtypeStruct((2, NP, D), jnp.float32)],
    mesh=_mesh,
    scratch_types=[
        pltpu.VMEM((NCH, CH), jnp.int32),
        pltpu.VMEM((NCH, CH), jnp.int32),
        pltpu.VMEM((NCH, CH), jnp.int32),
        pltpu.VMEM((NCH, CH), jnp.int32),
        pltpu.VMEM((CH, D), jnp.float32),
        pltpu.SemaphoreType.DMA,
        pltpu.VMEM_SHARED((NP, D), jnp.float32),
        pltpu.VMEM_SHARED((NP, D), jnp.float32),
    ],
)
def _conv_kernel(gp_hbm, gn_hbm, z_hbm, sp_hbm, dp_hbm, sn_hbm, dn_hbm,
                 yp_hbm, yn_hbm, spv, dpv, snv, dnv, rows, sem, accp, accn):
    cid = lax.axis_index("c")
    sid = lax.axis_index("s")
    wid = cid * 16 + sid
    st = NP // 16
    pltpu.sync_copy(z_hbm.at[pl.ds(sid * st, st)], accp.at[pl.ds(sid * st, st)])
    pltpu.sync_copy(z_hbm.at[pl.ds(sid * st, st)], accn.at[pl.ds(sid * st, st)])
    pltpu.sync_copy(sp_hbm.at[wid], spv)
    pltpu.sync_copy(dp_hbm.at[wid], dpv)
    pltpu.sync_copy(sn_hbm.at[wid], snv)
    pltpu.sync_copy(dn_hbm.at[wid], dnv)
    plsc.subcore_barrier()

    def one_sign(g_hbm, sv, dv, acc):
        def body(j, carry):
            pltpu.async_copy(g_hbm.at[sv.at[j]], rows, sem).wait()
            pltpu.sync_copy(rows, acc.at[dv.at[j]], add=True)
            return carry
        lax.fori_loop(0, NCH, body, 0)

    one_sign(gp_hbm, spv, dpv, accp)
    one_sign(gn_hbm, snv, dnv, accn)
    plsc.subcore_barrier()
    pltpu.sync_copy(accp.at[pl.ds(sid * st, st)],
                    yp_hbm.at[cid, pl.ds(sid * st, st)])
    pltpu.sync_copy(accn.at[pl.ds(sid * st, st)],
                    yn_hbm.at[cid, pl.ds(sid * st, st)])


_GRID = 8
_BR = NP // _GRID


def _dense1_body(x_ref, w1p_ref, w1n_ref, hist_ref,
                 gp_ref, gn_ref, hp_ref, hn_ref, dp_ref, dn_ref):
    hist = hist_ref[...]
    disp = lax.rsqrt(hist[0, 0] + hist[1, 0] + 1.0)
    disn = lax.rsqrt(hist[0, 1] + hist[1, 1] + 1.0)
    hp = jnp.dot(x_ref[...], w1p_ref[...], preferred_element_type=jnp.float32)
    hn = jnp.dot(x_ref[...], w1n_ref[...], preferred_element_type=jnp.float32)
    gp_ref[...] = hp * disp
    gn_ref[...] = hn * disn
    hp_ref[...] = hp
    hn_ref[...] = hn
    dp_ref[...] = disp
    dn_ref[...] = disn


_dense1 = pl.pallas_call(
    _dense1_body,
    grid=(_GRID,),
    in_specs=[
        pl.BlockSpec((_BR, F), lambda i: (i, 0)),
        pl.BlockSpec((F, D), lambda i: (0, 0)),
        pl.BlockSpec((F, D), lambda i: (0, 0)),
        pl.BlockSpec((2, 2, _BR, 1), lambda i: (0, 0, i, 0)),
    ],
    out_specs=[
        pl.BlockSpec((_BR, D), lambda i: (i, 0)),
        pl.BlockSpec((_BR, D), lambda i: (i, 0)),
        pl.BlockSpec((_BR, D), lambda i: (i, 0)),
        pl.BlockSpec((_BR, D), lambda i: (i, 0)),
        pl.BlockSpec((_BR, 1), lambda i: (i, 0)),
        pl.BlockSpec((_BR, 1), lambda i: (i, 0)),
    ],
    out_shape=[jax.ShapeDtypeStruct((NP, D), jnp.float32)] * 4
    + [jax.ShapeDtypeStruct((NP, 1), jnp.float32)] * 2,
)


def _dense2_body(ypp_ref, ynp_ref, hp_ref, hn_ref, dp_ref, dn_ref,
                 b1p_ref, b1n_ref, w2p_ref, w2n_ref,
                 gp2_ref, gn2_ref, hp2_ref, hn2_ref):
    disp = dp_ref[...]
    disn = dn_ref[...]
    yp = ypp_ref[0] + ypp_ref[1]
    yn = ynp_ref[0] + ynp_ref[1]
    ap = jnp.maximum(disp * yp + disp * disp * hp_ref[...] + b1p_ref[...], 0.0)
    an = jnp.maximum(disn * yn + disn * disn * hn_ref[...] + b1n_ref[...], 0.0)
    h = ap - an
    hp2 = jnp.dot(h, w2p_ref[...], preferred_element_type=jnp.float32)
    hn2 = jnp.dot(h, w2n_ref[...], preferred_element_type=jnp.float32)
    hp2_ref[...] = hp2
    hn2_ref[...] = hn2
    gp2_ref[...] = hp2 * disp
    gn2_ref[...] = hn2 * disn


_dense2 = pl.pallas_call(
    _dense2_body,
    grid=(_GRID,),
    in_specs=[
        pl.BlockSpec((2, _BR, D), lambda i: (0, i, 0)),
        pl.BlockSpec((2, _BR, D), lambda i: (0, i, 0)),
        pl.BlockSpec((_BR, D), lambda i: (i, 0)),
        pl.BlockSpec((_BR, D), lambda i: (i, 0)),
        pl.BlockSpec((_BR, 1), lambda i: (i, 0)),
        pl.BlockSpec((_BR, 1), lambda i: (i, 0)),
        pl.BlockSpec((1, D), lambda i: (0, 0)),
        pl.BlockSpec((1, D), lambda i: (0, 0)),
        pl.BlockSpec((D, D), lambda i: (0, 0)),
        pl.BlockSpec((D, D), lambda i: (0, 0)),
    ],
    out_specs=[pl.BlockSpec((_BR, D), lambda i: (i, 0))] * 4,
    out_shape=[jax.ShapeDtypeStruct((NP, D), jnp.float32)] * 4,
)


def _dense3_body(ypp_ref, ynp_ref, hp2_ref, hn2_ref, dp_ref, dn_ref,
                 b2p_ref, b2n_ref, out_ref):
    disp = dp_ref[...]
    disn = dn_ref[...]
    yp = ypp_ref[0] + ypp_ref[1]
    yn = ynp_ref[0] + ynp_ref[1]
    op = jnp.maximum(disp * yp + disp * disp * hp2_ref[...] + b2p_ref[...], 0.0)
    on = jnp.maximum(disn * yn + disn * disn * hn2_ref[...] + b2n_ref[...], 0.0)
    o = op - on
    m = jnp.max(o, axis=1, keepdims=True)
    lse = jnp.log(jnp.sum(jnp.exp(o - m), axis=1, keepdims=True)) + m
    out_ref[...] = o - lse


_dense3 = pl.pallas_call(
    _dense3_body,
    grid=(_GRID,),
    in_specs=[
        pl.BlockSpec((2, _BR, D), lambda i: (0, i, 0)),
        pl.BlockSpec((2, _BR, D), lambda i: (0, i, 0)),
        pl.BlockSpec((_BR, D), lambda i: (i, 0)),
        pl.BlockSpec((_BR, D), lambda i: (i, 0)),
        pl.BlockSpec((_BR, 1), lambda i: (i, 0)),
        pl.BlockSpec((_BR, 1), lambda i: (i, 0)),
        pl.BlockSpec((1, D), lambda i: (0, 0)),
        pl.BlockSpec((1, D), lambda i: (0, 0)),
    ],
    out_specs=pl.BlockSpec((_BR, D), lambda i: (i, 0)),
    out_shape=jax.ShapeDtypeStruct((NP, D), jnp.float32),
)


def _pad_edges(v):
    v = v.astype(jnp.int32)
    return jnp.concatenate([v, jnp.full((EP - E,), NP - 1, jnp.int32)])


def kernel(x, edge_index_pos, edge_index_neg,
           W1p, b1p, W1n, b1n, W2p, b2p, W2n, b2n):
    sp = _pad_edges(edge_index_pos[0])
    dp = _pad_edges(edge_index_pos[1])
    sn = _pad_edges(edge_index_neg[0])
    dn = _pad_edges(edge_index_neg[1])
    sp_r = sp.reshape(NW, NCH, CH)
    dp_r = dp.reshape(NW, NCH, CH)
    sn_r = sn.reshape(NW, NCH, CH)
    dn_r = dn.reshape(NW, NCH, CH)
    xp = jnp.pad(x, ((0, NP - N_NODES), (0, 0)))
    zeros_tbl = jnp.zeros((NP, D), jnp.float32)

    hist = _deg_kernel(dp.reshape(NW, EW), dn.reshape(NW, EW), zeros_tbl[:NF])
    gp, gn, hp, hn, disp, disn = _dense1(
        xp, W1p, W1n, hist.reshape(2, 2, NP, 1))
    ypp, ynp = _conv_kernel(gp, gn, zeros_tbl, sp_r, dp_r, sn_r, dn_r)
    gp2, gn2, hp2, hn2 = _dense2(ypp, ynp, hp, hn, disp, disn,
                                 b1p.reshape(1, D), b1n.reshape(1, D),
                                 W2p, W2n)
    ypp2, ynp2 = _conv_kernel(gp2, gn2, sp_r, dp_r, sn_r, dn_r) if False else \
        _conv_kernel(gp2, gn2, zeros_tbl, sp_r, dp_r, sn_r, dn_r)
    o = _dense3(ypp2, ynp2, hp2, hn2, disp, disn,
                b2p.reshape(1, D), b2n.reshape(1, D))
    return o[:N_NODES]


# speedup vs baseline: 33.0076x; 33.0076x over previous
import functools

import jax
import jax.numpy as jnp
from jax import lax
from jax.experimental import pallas as pl
from jax.experimental.pallas import tpu as pltpu
from jax.experimental.pallas import tpu_sc as plsc

N_NODES = 10000
NP = 10240
D = 16
F = 128
E = 320000
NW = 32
CH = 128
NCH = 81
EW = NCH * CH
EP = NW * EW

_mesh = plsc.VectorSubcoreMesh(core_axis_name="c", subcore_axis_name="s")
_sc_params = pltpu.CompilerParams(use_tc_tiling_on_sc=False)


@functools.partial(
    pl.kernel,
    out_type=jax.ShapeDtypeStruct((2, 2, NP, D), jnp.float32),
    mesh=_mesh,
    scratch_types=[
        pltpu.VMEM((NCH, CH), jnp.int32),
        pltpu.VMEM((NCH, CH), jnp.int32),
        pltpu.VMEM((CH, D), jnp.float32),
        pltpu.VMEM_SHARED((NP, D), jnp.float32),
        pltpu.VMEM_SHARED((NP, D), jnp.float32),
    ],
    compiler_params=_sc_params,
)
def _deg_kernel(dp_hbm, dn_hbm, z_hbm, ones_hbm, out_hbm,
                dpv, dnv, ones_v, accp, accn):
    cid = lax.axis_index("c")
    sid = lax.axis_index("s")
    wid = cid * 16 + sid
    st = NP // 16
    pltpu.sync_copy(z_hbm.at[pl.ds(sid * st, st)], accp.at[pl.ds(sid * st, st)])
    pltpu.sync_copy(z_hbm.at[pl.ds(sid * st, st)], accn.at[pl.ds(sid * st, st)])
    pltpu.sync_copy(ones_hbm, ones_v)
    pltpu.sync_copy(dp_hbm.at[wid], dpv)
    pltpu.sync_copy(dn_hbm.at[wid], dnv)
    plsc.subcore_barrier()

    def one_sign(dv, acc):
        def body(j, carry):
            pltpu.sync_copy(ones_v, acc.at[dv.at[j]], add=True)
            return carry
        lax.fori_loop(0, NCH, body, 0)

    one_sign(dpv, accp)
    one_sign(dnv, accn)
    plsc.subcore_barrier()
    pltpu.sync_copy(accp.at[pl.ds(sid * st, st)],
                    out_hbm.at[cid, 0, pl.ds(sid * st, st)])
    pltpu.sync_copy(accn.at[pl.ds(sid * st, st)],
                    out_hbm.at[cid, 1, pl.ds(sid * st, st)])


@functools.partial(
    pl.kernel,
    out_type=[jax.ShapeDtypeStruct((2, NP, D), jnp.float32),
              jax.ShapeDtypeStruct((2, NP, D), jnp.float32)],
    mesh=_mesh,
    scratch_types=[
        pltpu.VMEM((NCH, CH), jnp.int32),
        pltpu.VMEM((NCH, CH), jnp.int32),
        pltpu.VMEM((NCH, CH), jnp.int32),
        pltpu.VMEM((NCH, CH), jnp.int32),
        pltpu.VMEM((CH, D), jnp.float32),
        pltpu.SemaphoreType.DMA,
        pltpu.VMEM_SHARED((NP, D), jnp.float32),
        pltpu.VMEM_SHARED((NP, D), jnp.float32),
    ],
    compiler_params=_sc_params,
)
def _conv_kernel(gp_hbm, gn_hbm, z_hbm, sp_hbm, dp_hbm, sn_hbm, dn_hbm,
                 yp_hbm, yn_hbm, spv, dpv, snv, dnv, rows, sem, accp, accn):
    cid = lax.axis_index("c")
    sid = lax.axis_index("s")
    wid = cid * 16 + sid
    st = NP // 16
    pltpu.sync_copy(z_hbm.at[pl.ds(sid * st, st)], accp.at[pl.ds(sid * st, st)])
    pltpu.sync_copy(z_hbm.at[pl.ds(sid * st, st)], accn.at[pl.ds(sid * st, st)])
    pltpu.sync_copy(sp_hbm.at[wid], spv)
    pltpu.sync_copy(dp_hbm.at[wid], dpv)
    pltpu.sync_copy(sn_hbm.at[wid], snv)
    pltpu.sync_copy(dn_hbm.at[wid], dnv)
    plsc.subcore_barrier()

    def one_sign(g_hbm, sv, dv, acc):
        def body(j, carry):
            pltpu.async_copy(g_hbm.at[sv.at[j]], rows, sem).wait()
            pltpu.sync_copy(rows, acc.at[dv.at[j]], add=True)
            return carry
        lax.fori_loop(0, NCH, body, 0)

    one_sign(gp_hbm, spv, dpv, accp)
    one_sign(gn_hbm, snv, dnv, accn)
    plsc.subcore_barrier()
    pltpu.sync_copy(accp.at[pl.ds(sid * st, st)],
                    yp_hbm.at[cid, pl.ds(sid * st, st)])
    pltpu.sync_copy(accn.at[pl.ds(sid * st, st)],
                    yn_hbm.at[cid, pl.ds(sid * st, st)])


_GRID = 8
_BR = NP // _GRID


def _dis(deg):
    return jnp.where(deg > 0.0, lax.rsqrt(deg), 0.0)


def _dense1_body(x_ref, w1p_ref, w1n_ref, deg_ref,
                 gp_ref, gn_ref, dp_ref, dn_ref):
    deg = deg_ref[...]
    disp = _dis(deg[0, 0] + deg[1, 0])
    disn = _dis(deg[0, 1] + deg[1, 1])
    hp = jnp.dot(x_ref[...], w1p_ref[...], preferred_element_type=jnp.float32)
    hn = jnp.dot(x_ref[...], w1n_ref[...], preferred_element_type=jnp.float32)
    gp_ref[...] = hp * disp
    gn_ref[...] = hn * disn
    dp_ref[...] = disp
    dn_ref[...] = disn


_dense1 = pl.pallas_call(
    _dense1_body,
    grid=(_GRID,),
    in_specs=[
        pl.BlockSpec((_BR, F), lambda i: (i, 0)),
        pl.BlockSpec((F, D), lambda i: (0, 0)),
        pl.BlockSpec((F, D), lambda i: (0, 0)),
        pl.BlockSpec((2, 2, _BR, D), lambda i: (0, 0, i, 0)),
    ],
    out_specs=[pl.BlockSpec((_BR, D), lambda i: (i, 0))] * 4,
    out_shape=[jax.ShapeDtypeStruct((NP, D), jnp.float32)] * 4,
)


def _dense2_body(ypp_ref, ynp_ref, dp_ref, dn_ref,
                 b1p_ref, b1n_ref, w2p_ref, w2n_ref,
                 gp2_ref, gn2_ref):
    disp = dp_ref[...]
    disn = dn_ref[...]
    yp = ypp_ref[0] + ypp_ref[1]
    yn = ynp_ref[0] + ynp_ref[1]
    ap = jnp.maximum(disp * yp + b1p_ref[...], 0.0)
    an = jnp.maximum(disn * yn + b1n_ref[...], 0.0)
    h = ap - an
    hp2 = jnp.dot(h, w2p_ref[...], preferred_element_type=jnp.float32)
    hn2 = jnp.dot(h, w2n_ref[...], preferred_element_type=jnp.float32)
    gp2_ref[...] = hp2 * disp
    gn2_ref[...] = hn2 * disn


_dense2 = pl.pallas_call(
    _dense2_body,
    grid=(_GRID,),
    in_specs=[
        pl.BlockSpec((2, _BR, D), lambda i: (0, i, 0)),
        pl.BlockSpec((2, _BR, D), lambda i: (0, i, 0)),
        pl.BlockSpec((_BR, D), lambda i: (i, 0)),
        pl.BlockSpec((_BR, D), lambda i: (i, 0)),
        pl.BlockSpec((1, D), lambda i: (0, 0)),
        pl.BlockSpec((1, D), lambda i: (0, 0)),
        pl.BlockSpec((D, D), lambda i: (0, 0)),
        pl.BlockSpec((D, D), lambda i: (0, 0)),
    ],
    out_specs=[pl.BlockSpec((_BR, D), lambda i: (i, 0))] * 2,
    out_shape=[jax.ShapeDtypeStruct((NP, D), jnp.float32)] * 2,
)


def _dense3_body(ypp_ref, ynp_ref, dp_ref, dn_ref,
                 b2p_ref, b2n_ref, out_ref):
    disp = dp_ref[...]
    disn = dn_ref[...]
    yp = ypp_ref[0] + ypp_ref[1]
    yn = ynp_ref[0] + ynp_ref[1]
    op = jnp.maximum(disp * yp + b2p_ref[...], 0.0)
    on = jnp.maximum(disn * yn + b2n_ref[...], 0.0)
    o = op - on
    m = jnp.max(o, axis=1, keepdims=True)
    lse = jnp.log(jnp.sum(jnp.exp(o - m), axis=1, keepdims=True)) + m
    out_ref[...] = o - lse


_dense3 = pl.pallas_call(
    _dense3_body,
    grid=(_GRID,),
    in_specs=[
        pl.BlockSpec((2, _BR, D), lambda i: (0, i, 0)),
        pl.BlockSpec((2, _BR, D), lambda i: (0, i, 0)),
        pl.BlockSpec((_BR, D), lambda i: (i, 0)),
        pl.BlockSpec((_BR, D), lambda i: (i, 0)),
        pl.BlockSpec((1, D), lambda i: (0, 0)),
        pl.BlockSpec((1, D), lambda i: (0, 0)),
    ],
    out_specs=pl.BlockSpec((_BR, D), lambda i: (i, 0)),
    out_shape=jax.ShapeDtypeStruct((NP, D), jnp.float32),
)


def _pad_edges(v, loop):
    v = v.astype(jnp.int32)
    return jnp.concatenate(
        [v, loop, jnp.full((EP - E - N_NODES,), NP - 1, jnp.int32)]
    ).reshape(NW, NCH, CH)


def kernel(x, edge_index_pos, edge_index_neg,
           W1p, b1p, W1n, b1n, W2p, b2p, W2n, b2n):
    loop = jnp.arange(N_NODES, dtype=jnp.int32)
    sp_r = _pad_edges(edge_index_pos[0], loop)
    dp_r = _pad_edges(edge_index_pos[1], loop)
    sn_r = _pad_edges(edge_index_neg[0], loop)
    dn_r = _pad_edges(edge_index_neg[1], loop)
    xp = jnp.pad(x, ((0, NP - N_NODES), (0, 0)))
    zeros_tbl = jnp.zeros((NP, D), jnp.float32)
    ones_tbl = jnp.ones((CH, D), jnp.float32)

    deg = _deg_kernel(dp_r, dn_r, zeros_tbl, ones_tbl)
    gp, gn, disp, disn = _dense1(xp, W1p, W1n, deg)
    ypp, ynp = _conv_kernel(gp, gn, zeros_tbl, sp_r, dp_r, sn_r, dn_r)
    gp2, gn2 = _dense2(ypp, ynp, disp, disn,
                       b1p.reshape(1, D), b1n.reshape(1, D), W2p, W2n)
    ypp2, ynp2 = _conv_kernel(gp2, gn2, zeros_tbl, sp_r, dp_r, sn_r, dn_r)
    o = _dense3(ypp2, ynp2, disp, disn,
                b2p.reshape(1, D), b2n.reshape(1, D))
    return o[:N_NODES]


# speedup vs baseline: 49.6561x; 1.5044x over previous
import functools

import jax
import jax.numpy as jnp
from jax import lax
from jax.experimental import pallas as pl
from jax.experimental.pallas import tpu as pltpu
from jax.experimental.pallas import tpu_sc as plsc

N_NODES = 10000
NP = 10240
D = 16
F = 128
E = 320000
NW = 32
CH = 128
NCH = 81
KG = 9
EW = NCH * CH
EP = NW * EW

_mesh = plsc.VectorSubcoreMesh(core_axis_name="c", subcore_axis_name="s")
_sc_params = pltpu.CompilerParams(use_tc_tiling_on_sc=False)


@functools.partial(
    pl.kernel,
    out_type=jax.ShapeDtypeStruct((2, 2, NP, D), jnp.float32),
    mesh=_mesh,
    scratch_types=[
        pltpu.VMEM((NCH, CH), jnp.int32),
        pltpu.VMEM((NCH, CH), jnp.int32),
        pltpu.VMEM((CH, D), jnp.float32),
        pltpu.SemaphoreType.DMA,
        pltpu.VMEM_SHARED((NP, D), jnp.float32),
        pltpu.VMEM_SHARED((NP, D), jnp.float32),
    ],
    compiler_params=_sc_params,
)
def _deg_kernel(dp_hbm, dn_hbm, z_hbm, ones_hbm, out_hbm,
                dpv, dnv, ones_v, sem_s, accp, accn):
    cid = lax.axis_index("c")
    sid = lax.axis_index("s")
    wid = cid * 16 + sid
    st = NP // 16
    pltpu.sync_copy(z_hbm.at[pl.ds(sid * st, st)], accp.at[pl.ds(sid * st, st)])
    pltpu.sync_copy(z_hbm.at[pl.ds(sid * st, st)], accn.at[pl.ds(sid * st, st)])
    pltpu.sync_copy(ones_hbm, ones_v)
    pltpu.sync_copy(dp_hbm.at[wid], dpv)
    pltpu.sync_copy(dn_hbm.at[wid], dnv)
    plsc.subcore_barrier()

    def one_sign(dv, acc):
        def fire(j, carry):
            pltpu.async_copy(ones_v, acc.at[dv.at[j]], sem_s, add=True)
            return carry
        lax.fori_loop(0, NCH, fire, 0)

        def drain(j, carry):
            pltpu.make_async_copy(ones_v, acc.at[dv.at[0]], sem_s).wait()
            return carry
        lax.fori_loop(0, NCH, drain, 0)

    one_sign(dpv, accp)
    one_sign(dnv, accn)
    plsc.subcore_barrier()
    pltpu.sync_copy(accp.at[pl.ds(sid * st, st)],
                    out_hbm.at[cid, 0, pl.ds(sid * st, st)])
    pltpu.sync_copy(accn.at[pl.ds(sid * st, st)],
                    out_hbm.at[cid, 1, pl.ds(sid * st, st)])


@functools.partial(
    pl.kernel,
    out_type=[jax.ShapeDtypeStruct((2, NP, D), jnp.float32),
              jax.ShapeDtypeStruct((2, NP, D), jnp.float32)],
    mesh=_mesh,
    scratch_types=[
        pltpu.VMEM((NCH, CH), jnp.int32),
        pltpu.VMEM((NCH, CH), jnp.int32),
        pltpu.VMEM((NCH, CH), jnp.int32),
        pltpu.VMEM((NCH, CH), jnp.int32),
        pltpu.VMEM((KG, CH, D), jnp.float32),
        pltpu.SemaphoreType.DMA,
        pltpu.SemaphoreType.DMA,
        pltpu.VMEM_SHARED((NP, D), jnp.float32),
        pltpu.VMEM_SHARED((NP, D), jnp.float32),
    ],
    compiler_params=_sc_params,
)
def _conv_kernel(gp_hbm, gn_hbm, z_hbm, sp_hbm, dp_hbm, sn_hbm, dn_hbm,
                 yp_hbm, yn_hbm, spv, dpv, snv, dnv, rows, sem_g, sem_s,
                 accp, accn):
    cid = lax.axis_index("c")
    sid = lax.axis_index("s")
    wid = cid * 16 + sid
    st = NP // 16
    pltpu.sync_copy(z_hbm.at[pl.ds(sid * st, st)], accp.at[pl.ds(sid * st, st)])
    pltpu.sync_copy(z_hbm.at[pl.ds(sid * st, st)], accn.at[pl.ds(sid * st, st)])
    pltpu.sync_copy(sp_hbm.at[wid], spv)
    pltpu.sync_copy(dp_hbm.at[wid], dpv)
    pltpu.sync_copy(sn_hbm.at[wid], snv)
    pltpu.sync_copy(dn_hbm.at[wid], dnv)
    plsc.subcore_barrier()

    def one_sign(g_hbm, sv, dv, acc):
        def group(t, carry):
            base = t * KG
            for k in range(KG):
                pltpu.async_copy(g_hbm.at[sv.at[base + k]], rows.at[k], sem_g)
            for k in range(KG):
                pltpu.make_async_copy(g_hbm.at[sv.at[base + k]],
                                      rows.at[k], sem_g).wait()
                pltpu.async_copy(rows.at[k], acc.at[dv.at[base + k]],
                                 sem_s, add=True)
            for k in range(KG):
                pltpu.make_async_copy(rows.at[k], acc.at[dv.at[base + k]],
                                      sem_s).wait()
            return carry
        lax.fori_loop(0, NCH // KG, group, 0)

    one_sign(gp_hbm, spv, dpv, accp)
    one_sign(gn_hbm, snv, dnv, accn)
    plsc.subcore_barrier()
    pltpu.sync_copy(accp.at[pl.ds(sid * st, st)],
                    yp_hbm.at[cid, pl.ds(sid * st, st)])
    pltpu.sync_copy(accn.at[pl.ds(sid * st, st)],
                    yn_hbm.at[cid, pl.ds(sid * st, st)])


_GRID = 8
_BR = NP // _GRID


def _dis(deg):
    return jnp.where(deg > 0.0, lax.rsqrt(deg), 0.0)


def _dense1_body(x_ref, w1p_ref, w1n_ref, deg_ref,
                 gp_ref, gn_ref, dp_ref, dn_ref):
    deg = deg_ref[...]
    disp = _dis(deg[0, 0] + deg[1, 0])
    disn = _dis(deg[0, 1] + deg[1, 1])
    hp = jnp.dot(x_ref[...], w1p_ref[...], preferred_element_type=jnp.float32)
    hn = jnp.dot(x_ref[...], w1n_ref[...], preferred_element_type=jnp.float32)
    gp_ref[...] = hp * disp
    gn_ref[...] = hn * disn
    dp_ref[...] = disp
    dn_ref[...] = disn


_dense1 = pl.pallas_call(
    _dense1_body,
    grid=(_GRID,),
    in_specs=[
        pl.BlockSpec((_BR, F), lambda i: (i, 0)),
        pl.BlockSpec((F, D), lambda i: (0, 0)),
        pl.BlockSpec((F, D), lambda i: (0, 0)),
        pl.BlockSpec((2, 2, _BR, D), lambda i: (0, 0, i, 0)),
    ],
    out_specs=[pl.BlockSpec((_BR, D), lambda i: (i, 0))] * 4,
    out_shape=[jax.ShapeDtypeStruct((NP, D), jnp.float32)] * 4,
)


def _dense2_body(ypp_ref, ynp_ref, dp_ref, dn_ref,
                 b1p_ref, b1n_ref, w2p_ref, w2n_ref,
                 gp2_ref, gn2_ref):
    disp = dp_ref[...]
    disn = dn_ref[...]
    yp = ypp_ref[0] + ypp_ref[1]
    yn = ynp_ref[0] + ynp_ref[1]
    ap = jnp.maximum(disp * yp + b1p_ref[...], 0.0)
    an = jnp.maximum(disn * yn + b1n_ref[...], 0.0)
    h = ap - an
    hp2 = jnp.dot(h, w2p_ref[...], preferred_element_type=jnp.float32)
    hn2 = jnp.dot(h, w2n_ref[...], preferred_element_type=jnp.float32)
    gp2_ref[...] = hp2 * disp
    gn2_ref[...] = hn2 * disn


_dense2 = pl.pallas_call(
    _dense2_body,
    grid=(_GRID,),
    in_specs=[
        pl.BlockSpec((2, _BR, D), lambda i: (0, i, 0)),
        pl.BlockSpec((2, _BR, D), lambda i: (0, i, 0)),
        pl.BlockSpec((_BR, D), lambda i: (i, 0)),
        pl.BlockSpec((_BR, D), lambda i: (i, 0)),
        pl.BlockSpec((1, D), lambda i: (0, 0)),
        pl.BlockSpec((1, D), lambda i: (0, 0)),
        pl.BlockSpec((D, D), lambda i: (0, 0)),
        pl.BlockSpec((D, D), lambda i: (0, 0)),
    ],
    out_specs=[pl.BlockSpec((_BR, D), lambda i: (i, 0))] * 2,
    out_shape=[jax.ShapeDtypeStruct((NP, D), jnp.float32)] * 2,
)


def _dense3_body(ypp_ref, ynp_ref, dp_ref, dn_ref,
                 b2p_ref, b2n_ref, out_ref):
    disp = dp_ref[...]
    disn = dn_ref[...]
    yp = ypp_ref[0] + ypp_ref[1]
    yn = ynp_ref[0] + ynp_ref[1]
    op = jnp.maximum(disp * yp + b2p_ref[...], 0.0)
    on = jnp.maximum(disn * yn + b2n_ref[...], 0.0)
    o = op - on
    m = jnp.max(o, axis=1, keepdims=True)
    lse = jnp.log(jnp.sum(jnp.exp(o - m), axis=1, keepdims=True)) + m
    out_ref[...] = o - lse


_dense3 = pl.pallas_call(
    _dense3_body,
    grid=(_GRID,),
    in_specs=[
        pl.BlockSpec((2, _BR, D), lambda i: (0, i, 0)),
        pl.BlockSpec((2, _BR, D), lambda i: (0, i, 0)),
        pl.BlockSpec((_BR, D), lambda i: (i, 0)),
        pl.BlockSpec((_BR, D), lambda i: (i, 0)),
        pl.BlockSpec((1, D), lambda i: (0, 0)),
        pl.BlockSpec((1, D), lambda i: (0, 0)),
    ],
    out_specs=pl.BlockSpec((_BR, D), lambda i: (i, 0)),
    out_shape=jax.ShapeDtypeStruct((NP, D), jnp.float32),
)


def _pad_edges(v, loop):
    v = v.astype(jnp.int32)
    return jnp.concatenate(
        [v, loop, jnp.full((EP - E - N_NODES,), NP - 1, jnp.int32)]
    ).reshape(NW, NCH, CH)


def kernel(x, edge_index_pos, edge_index_neg,
           W1p, b1p, W1n, b1n, W2p, b2p, W2n, b2n):
    loop = jnp.arange(N_NODES, dtype=jnp.int32)
    sp_r = _pad_edges(edge_index_pos[0], loop)
    dp_r = _pad_edges(edge_index_pos[1], loop)
    sn_r = _pad_edges(edge_index_neg[0], loop)
    dn_r = _pad_edges(edge_index_neg[1], loop)
    xp = jnp.pad(x, ((0, NP - N_NODES), (0, 0)))
    zeros_tbl = jnp.zeros((NP, D), jnp.float32)
    ones_tbl = jnp.ones((CH, D), jnp.float32)

    deg = _deg_kernel(dp_r, dn_r, zeros_tbl, ones_tbl)
    gp, gn, disp, disn = _dense1(xp, W1p, W1n, deg)
    ypp, ynp = _conv_kernel(gp, gn, zeros_tbl, sp_r, dp_r, sn_r, dn_r)
    gp2, gn2 = _dense2(ypp, ynp, disp, disn,
                       b1p.reshape(1, D), b1n.reshape(1, D), W2p, W2n)
    ypp2, ynp2 = _conv_kernel(gp2, gn2, zeros_tbl, sp_r, dp_r, sn_r, dn_r)
    o = _dense3(ypp2, ynp2, disp, disn,
                b2p.reshape(1, D), b2n.reshape(1, D))
    return o[:N_NODES]


# speedup vs baseline: 51.8694x; 1.0446x over previous
import functools

import jax
import jax.numpy as jnp
from jax import lax
from jax.experimental import pallas as pl
from jax.experimental.pallas import tpu as pltpu
from jax.experimental.pallas import tpu_sc as plsc

N_NODES = 10000
NP = 10240
D = 16
F = 128
E = 320000
NW = 32
CH = 128
NCH = 81
KG = 9
EW = NCH * CH
EP = NW * EW

_mesh = plsc.VectorSubcoreMesh(core_axis_name="c", subcore_axis_name="s")
_sc_params = pltpu.CompilerParams(use_tc_tiling_on_sc=False)


@functools.partial(
    pl.kernel,
    out_type=jax.ShapeDtypeStruct((2, 2, NP, D), jnp.float32),
    mesh=_mesh,
    scratch_types=[
        pltpu.VMEM((NCH, CH), jnp.int32),
        pltpu.VMEM((NCH, CH), jnp.int32),
        pltpu.VMEM((CH, D), jnp.float32),
        pltpu.SemaphoreType.DMA,
        pltpu.VMEM_SHARED((NP, D), jnp.float32),
        pltpu.VMEM_SHARED((NP, D), jnp.float32),
    ],
    compiler_params=_sc_params,
)
def _deg_kernel(dp_hbm, dn_hbm, z_hbm, ones_hbm, out_hbm,
                dpv, dnv, ones_v, sem_s, accp, accn):
    cid = lax.axis_index("c")
    sid = lax.axis_index("s")
    wid = cid * 16 + sid
    st = NP // 16
    pltpu.sync_copy(z_hbm.at[pl.ds(sid * st, st)], accp.at[pl.ds(sid * st, st)])
    pltpu.sync_copy(z_hbm.at[pl.ds(sid * st, st)], accn.at[pl.ds(sid * st, st)])
    pltpu.sync_copy(ones_hbm, ones_v)
    pltpu.sync_copy(dp_hbm.at[wid], dpv)
    pltpu.sync_copy(dn_hbm.at[wid], dnv)
    plsc.subcore_barrier()

    def one_sign(dv, acc):
        def fire(j, carry):
            pltpu.async_copy(ones_v, acc.at[dv.at[j]], sem_s, add=True)
            return carry
        lax.fori_loop(0, NCH, fire, 0)

        def drain(j, carry):
            pltpu.make_async_copy(ones_v, acc.at[dv.at[0]], sem_s).wait()
            return carry
        lax.fori_loop(0, NCH, drain, 0)

    one_sign(dpv, accp)
    one_sign(dnv, accn)
    plsc.subcore_barrier()
    pltpu.sync_copy(accp.at[pl.ds(sid * st, st)],
                    out_hbm.at[cid, 0, pl.ds(sid * st, st)])
    pltpu.sync_copy(accn.at[pl.ds(sid * st, st)],
                    out_hbm.at[cid, 1, pl.ds(sid * st, st)])


@functools.partial(
    pl.kernel,
    out_type=[jax.ShapeDtypeStruct((2, NP, D), jnp.float32),
              jax.ShapeDtypeStruct((2, NP, D), jnp.float32)],
    mesh=_mesh,
    scratch_types=[
        pltpu.VMEM((NCH, CH), jnp.int32),
        pltpu.VMEM((NCH, CH), jnp.int32),
        pltpu.VMEM((NCH, CH), jnp.int32),
        pltpu.VMEM((NCH, CH), jnp.int32),
        pltpu.VMEM((2, KG, CH, D), jnp.float32),
        pltpu.SemaphoreType.DMA,
        pltpu.SemaphoreType.DMA,
        pltpu.VMEM_SHARED((NP, D), jnp.float32),
        pltpu.VMEM_SHARED((NP, D), jnp.float32),
    ],
    compiler_params=_sc_params,
)
def _conv_kernel(gp_hbm, gn_hbm, z_hbm, sp_hbm, dp_hbm, sn_hbm, dn_hbm,
                 yp_hbm, yn_hbm, spv, dpv, snv, dnv, rows, sem_g, sem_s,
                 accp, accn):
    cid = lax.axis_index("c")
    sid = lax.axis_index("s")
    wid = cid * 16 + sid
    st = NP // 16
    pltpu.sync_copy(z_hbm.at[pl.ds(sid * st, st)], accp.at[pl.ds(sid * st, st)])
    pltpu.sync_copy(z_hbm.at[pl.ds(sid * st, st)], accn.at[pl.ds(sid * st, st)])
    pltpu.sync_copy(sp_hbm.at[wid], spv)
    pltpu.sync_copy(dp_hbm.at[wid], dpv)
    pltpu.sync_copy(sn_hbm.at[wid], snv)
    pltpu.sync_copy(dn_hbm.at[wid], dnv)
    plsc.subcore_barrier()

    NG = NCH // KG

    def one_sign(g_hbm, sv, dv, acc):
        for k in range(KG):
            pltpu.async_copy(g_hbm.at[sv.at[k]], rows.at[0, k], sem_g)

        def group(t, carry):
            par = lax.rem(t, 2)
            nxt = 1 - par
            base = t * KG

            @pl.when(t + 1 < NG)
            def _fire_next():
                @pl.when(t >= 1)
                def _drain_prev_scatters():
                    for k in range(KG):
                        pltpu.make_async_copy(
                            rows.at[nxt, k],
                            acc.at[dv.at[base - KG + k]], sem_s).wait()
                for k in range(KG):
                    pltpu.async_copy(g_hbm.at[sv.at[base + KG + k]],
                                     rows.at[nxt, k], sem_g)

            for k in range(KG):
                pltpu.make_async_copy(g_hbm.at[sv.at[base + k]],
                                      rows.at[par, k], sem_g).wait()
                pltpu.async_copy(rows.at[par, k], acc.at[dv.at[base + k]],
                                 sem_s, add=True)
            return carry

        lax.fori_loop(0, NG, group, 0)
        for k in range(2 * KG):
            pltpu.make_async_copy(rows.at[0, 0], acc.at[dv.at[0]],
                                  sem_s).wait()

    one_sign(gp_hbm, spv, dpv, accp)
    one_sign(gn_hbm, snv, dnv, accn)
    plsc.subcore_barrier()
    pltpu.sync_copy(accp.at[pl.ds(sid * st, st)],
                    yp_hbm.at[cid, pl.ds(sid * st, st)])
    pltpu.sync_copy(accn.at[pl.ds(sid * st, st)],
                    yn_hbm.at[cid, pl.ds(sid * st, st)])


_GRID = 8
_BR = NP // _GRID


def _dis(deg):
    return jnp.where(deg > 0.0, lax.rsqrt(deg), 0.0)


def _dense1_body(x_ref, w1p_ref, w1n_ref, deg_ref,
                 gp_ref, gn_ref, dp_ref, dn_ref):
    deg = deg_ref[...]
    disp = _dis(deg[0, 0] + deg[1, 0])
    disn = _dis(deg[0, 1] + deg[1, 1])
    hp = jnp.dot(x_ref[...], w1p_ref[...], preferred_element_type=jnp.float32)
    hn = jnp.dot(x_ref[...], w1n_ref[...], preferred_element_type=jnp.float32)
    gp_ref[...] = hp * disp
    gn_ref[...] = hn * disn
    dp_ref[...] = disp
    dn_ref[...] = disn


_dense1 = pl.pallas_call(
    _dense1_body,
    grid=(_GRID,),
    in_specs=[
        pl.BlockSpec((_BR, F), lambda i: (i, 0)),
        pl.BlockSpec((F, D), lambda i: (0, 0)),
        pl.BlockSpec((F, D), lambda i: (0, 0)),
        pl.BlockSpec((2, 2, _BR, D), lambda i: (0, 0, i, 0)),
    ],
    out_specs=[pl.BlockSpec((_BR, D), lambda i: (i, 0))] * 4,
    out_shape=[jax.ShapeDtypeStruct((NP, D), jnp.float32)] * 4,
)


def _dense2_body(ypp_ref, ynp_ref, dp_ref, dn_ref,
                 b1p_ref, b1n_ref, w2p_ref, w2n_ref,
                 gp2_ref, gn2_ref):
    disp = dp_ref[...]
    disn = dn_ref[...]
    yp = ypp_ref[0] + ypp_ref[1]
    yn = ynp_ref[0] + ynp_ref[1]
    ap = jnp.maximum(disp * yp + b1p_ref[...], 0.0)
    an = jnp.maximum(disn * yn + b1n_ref[...], 0.0)
    h = ap - an
    hp2 = jnp.dot(h, w2p_ref[...], preferred_element_type=jnp.float32)
    hn2 = jnp.dot(h, w2n_ref[...], preferred_element_type=jnp.float32)
    gp2_ref[...] = hp2 * disp
    gn2_ref[...] = hn2 * disn


_dense2 = pl.pallas_call(
    _dense2_body,
    grid=(_GRID,),
    in_specs=[
        pl.BlockSpec((2, _BR, D), lambda i: (0, i, 0)),
        pl.BlockSpec((2, _BR, D), lambda i: (0, i, 0)),
        pl.BlockSpec((_BR, D), lambda i: (i, 0)),
        pl.BlockSpec((_BR, D), lambda i: (i, 0)),
        pl.BlockSpec((1, D), lambda i: (0, 0)),
        pl.BlockSpec((1, D), lambda i: (0, 0)),
        pl.BlockSpec((D, D), lambda i: (0, 0)),
        pl.BlockSpec((D, D), lambda i: (0, 0)),
    ],
    out_specs=[pl.BlockSpec((_BR, D), lambda i: (i, 0))] * 2,
    out_shape=[jax.ShapeDtypeStruct((NP, D), jnp.float32)] * 2,
)


def _dense3_body(ypp_ref, ynp_ref, dp_ref, dn_ref,
                 b2p_ref, b2n_ref, out_ref):
    disp = dp_ref[...]
    disn = dn_ref[...]
    yp = ypp_ref[0] + ypp_ref[1]
    yn = ynp_ref[0] + ynp_ref[1]
    op = jnp.maximum(disp * yp + b2p_ref[...], 0.0)
    on = jnp.maximum(disn * yn + b2n_ref[...], 0.0)
    o = op - on
    m = jnp.max(o, axis=1, keepdims=True)
    lse = jnp.log(jnp.sum(jnp.exp(o - m), axis=1, keepdims=True)) + m
    out_ref[...] = o - lse


_dense3 = pl.pallas_call(
    _dense3_body,
    grid=(_GRID,),
    in_specs=[
        pl.BlockSpec((2, _BR, D), lambda i: (0, i, 0)),
        pl.BlockSpec((2, _BR, D), lambda i: (0, i, 0)),
        pl.BlockSpec((_BR, D), lambda i: (i, 0)),
        pl.BlockSpec((_BR, D), lambda i: (i, 0)),
        pl.BlockSpec((1, D), lambda i: (0, 0)),
        pl.BlockSpec((1, D), lambda i: (0, 0)),
    ],
    out_specs=pl.BlockSpec((_BR, D), lambda i: (i, 0)),
    out_shape=jax.ShapeDtypeStruct((NP, D), jnp.float32),
)


def _pad_edges(v, loop):
    v = v.astype(jnp.int32)
    return jnp.concatenate(
        [v, loop, jnp.full((EP - E - N_NODES,), NP - 1, jnp.int32)]
    ).reshape(NW, NCH, CH)


def kernel(x, edge_index_pos, edge_index_neg,
           W1p, b1p, W1n, b1n, W2p, b2p, W2n, b2n):
    loop = jnp.arange(N_NODES, dtype=jnp.int32)
    sp_r = _pad_edges(edge_index_pos[0], loop)
    dp_r = _pad_edges(edge_index_pos[1], loop)
    sn_r = _pad_edges(edge_index_neg[0], loop)
    dn_r = _pad_edges(edge_index_neg[1], loop)
    xp = jnp.pad(x, ((0, NP - N_NODES), (0, 0)))
    zeros_tbl = jnp.zeros((NP, D), jnp.float32)
    ones_tbl = jnp.ones((CH, D), jnp.float32)

    deg = _deg_kernel(dp_r, dn_r, zeros_tbl, ones_tbl)
    gp, gn, disp, disn = _dense1(xp, W1p, W1n, deg)
    ypp, ynp = _conv_kernel(gp, gn, zeros_tbl, sp_r, dp_r, sn_r, dn_r)
    gp2, gn2 = _dense2(ypp, ynp, disp, disn,
                       b1p.reshape(1, D), b1n.reshape(1, D), W2p, W2n)
    ypp2, ynp2 = _conv_kernel(gp2, gn2, zeros_tbl, sp_r, dp_r, sn_r, dn_r)
    o = _dense3(ypp2, ynp2, disp, disn,
                b2p.reshape(1, D), b2n.reshape(1, D))
    return o[:N_NODES]


# speedup vs baseline: 51.9132x; 1.0008x over previous
import functools

import jax
import jax.numpy as jnp
from jax import lax
from jax.experimental import pallas as pl
from jax.experimental.pallas import tpu as pltpu
from jax.experimental.pallas import tpu_sc as plsc

N_NODES = 10000
NP = 10240
D = 16
F = 128
E = 320000
NW = 32
CH = 128
NCH = 81
KG = 9
EW = NCH * CH
EP = NW * EW

_mesh = plsc.VectorSubcoreMesh(core_axis_name="c", subcore_axis_name="s")
_sc_params = pltpu.CompilerParams(use_tc_tiling_on_sc=False,
                                  skip_device_barrier=True)
_tc_params = pltpu.CompilerParams(skip_device_barrier=True)


@functools.partial(
    pl.kernel,
    out_type=jax.ShapeDtypeStruct((2, 2, NP, D), jnp.float32),
    mesh=_mesh,
    scratch_types=[
        pltpu.VMEM((NCH, CH), jnp.int32),
        pltpu.VMEM((NCH, CH), jnp.int32),
        pltpu.VMEM((CH, D), jnp.float32),
        pltpu.SemaphoreType.DMA,
        pltpu.VMEM_SHARED((NP, D), jnp.float32),
        pltpu.VMEM_SHARED((NP, D), jnp.float32),
    ],
    compiler_params=_sc_params,
)
def _deg_kernel(dp_hbm, dn_hbm, z_hbm, ones_hbm, out_hbm,
                dpv, dnv, ones_v, sem_s, accp, accn):
    cid = lax.axis_index("c")
    sid = lax.axis_index("s")
    wid = cid * 16 + sid
    st = NP // 16
    pltpu.sync_copy(z_hbm.at[pl.ds(sid * st, st)], accp.at[pl.ds(sid * st, st)])
    pltpu.sync_copy(z_hbm.at[pl.ds(sid * st, st)], accn.at[pl.ds(sid * st, st)])
    pltpu.sync_copy(ones_hbm, ones_v)
    pltpu.sync_copy(dp_hbm.at[wid], dpv)
    pltpu.sync_copy(dn_hbm.at[wid], dnv)
    plsc.subcore_barrier()

    def one_sign(dv, acc):
        def fire(j, carry):
            pltpu.async_copy(ones_v, acc.at[dv.at[j]], sem_s, add=True)
            return carry
        lax.fori_loop(0, NCH, fire, 0)

        def drain(j, carry):
            pltpu.make_async_copy(ones_v, acc.at[dv.at[0]], sem_s).wait()
            return carry
        lax.fori_loop(0, NCH, drain, 0)

    one_sign(dpv, accp)
    one_sign(dnv, accn)
    plsc.subcore_barrier()
    pltpu.sync_copy(accp.at[pl.ds(sid * st, st)],
                    out_hbm.at[cid, 0, pl.ds(sid * st, st)])
    pltpu.sync_copy(accn.at[pl.ds(sid * st, st)],
                    out_hbm.at[cid, 1, pl.ds(sid * st, st)])


@functools.partial(
    pl.kernel,
    out_type=[jax.ShapeDtypeStruct((2, NP, D), jnp.float32),
              jax.ShapeDtypeStruct((2, NP, D), jnp.float32)],
    mesh=_mesh,
    scratch_types=[
        pltpu.VMEM((NCH, CH), jnp.int32),
        pltpu.VMEM((NCH, CH), jnp.int32),
        pltpu.VMEM((NCH, CH), jnp.int32),
        pltpu.VMEM((NCH, CH), jnp.int32),
        pltpu.VMEM((2, KG, CH, D), jnp.float32),
        pltpu.SemaphoreType.DMA,
        pltpu.SemaphoreType.DMA,
        pltpu.VMEM_SHARED((NP, D), jnp.float32),
        pltpu.VMEM_SHARED((NP, D), jnp.float32),
    ],
    compiler_params=_sc_params,
)
def _conv_kernel(gp_hbm, gn_hbm, z_hbm, sp_hbm, dp_hbm, sn_hbm, dn_hbm,
                 yp_hbm, yn_hbm, spv, dpv, snv, dnv, rows, sem_g, sem_s,
                 accp, accn):
    cid = lax.axis_index("c")
    sid = lax.axis_index("s")
    wid = cid * 16 + sid
    st = NP // 16
    pltpu.sync_copy(z_hbm.at[pl.ds(sid * st, st)], accp.at[pl.ds(sid * st, st)])
    pltpu.sync_copy(z_hbm.at[pl.ds(sid * st, st)], accn.at[pl.ds(sid * st, st)])
    pltpu.sync_copy(sp_hbm.at[wid], spv)
    pltpu.sync_copy(dp_hbm.at[wid], dpv)
    pltpu.sync_copy(sn_hbm.at[wid], snv)
    pltpu.sync_copy(dn_hbm.at[wid], dnv)
    plsc.subcore_barrier()

    NG = NCH // KG

    def one_sign(g_hbm, sv, dv, acc):
        for k in range(KG):
            pltpu.async_copy(g_hbm.at[sv.at[k]], rows.at[0, k], sem_g)

        def group(t, carry):
            par = lax.rem(t, 2)
            nxt = 1 - par
            base = t * KG

            @pl.when(t + 1 < NG)
            def _fire_next():
                @pl.when(t >= 1)
                def _drain_prev_scatters():
                    for k in range(KG):
                        pltpu.make_async_copy(
                            rows.at[nxt, k],
                            acc.at[dv.at[base - KG + k]], sem_s).wait()
                for k in range(KG):
                    pltpu.async_copy(g_hbm.at[sv.at[base + KG + k]],
                                     rows.at[nxt, k], sem_g)

            for k in range(KG):
                pltpu.make_async_copy(g_hbm.at[sv.at[base + k]],
                                      rows.at[par, k], sem_g).wait()
                pltpu.async_copy(rows.at[par, k], acc.at[dv.at[base + k]],
                                 sem_s, add=True)
            return carry

        lax.fori_loop(0, NG, group, 0)
        for k in range(2 * KG):
            pltpu.make_async_copy(rows.at[0, 0], acc.at[dv.at[0]],
                                  sem_s).wait()

    one_sign(gp_hbm, spv, dpv, accp)
    one_sign(gn_hbm, snv, dnv, accn)
    plsc.subcore_barrier()
    pltpu.sync_copy(accp.at[pl.ds(sid * st, st)],
                    yp_hbm.at[cid, pl.ds(sid * st, st)])
    pltpu.sync_copy(accn.at[pl.ds(sid * st, st)],
                    yn_hbm.at[cid, pl.ds(sid * st, st)])


_GRID = 8
_BR = NP // _GRID


def _dis(deg):
    return jnp.where(deg > 0.0, lax.rsqrt(deg), 0.0)


def _dense1_body(x_ref, w1p_ref, w1n_ref, deg_ref,
                 gp_ref, gn_ref, dp_ref, dn_ref):
    deg = deg_ref[...]
    disp = _dis(deg[0, 0] + deg[1, 0])
    disn = _dis(deg[0, 1] + deg[1, 1])
    hp = jnp.dot(x_ref[...], w1p_ref[...], preferred_element_type=jnp.float32)
    hn = jnp.dot(x_ref[...], w1n_ref[...], preferred_element_type=jnp.float32)
    gp_ref[...] = hp * disp
    gn_ref[...] = hn * disn
    dp_ref[...] = disp
    dn_ref[...] = disn


_dense1 = pl.pallas_call(
    _dense1_body,
    grid=(_GRID,),
    in_specs=[
        pl.BlockSpec((_BR, F), lambda i: (i, 0)),
        pl.BlockSpec((F, D), lambda i: (0, 0)),
        pl.BlockSpec((F, D), lambda i: (0, 0)),
        pl.BlockSpec((2, 2, _BR, D), lambda i: (0, 0, i, 0)),
    ],
    out_specs=[pl.BlockSpec((_BR, D), lambda i: (i, 0))] * 4,
    out_shape=[jax.ShapeDtypeStruct((NP, D), jnp.float32)] * 4,
    compiler_params=_tc_params,
)


def _dense2_body(ypp_ref, ynp_ref, dp_ref, dn_ref,
                 b1p_ref, b1n_ref, w2p_ref, w2n_ref,
                 gp2_ref, gn2_ref):
    disp = dp_ref[...]
    disn = dn_ref[...]
    yp = ypp_ref[0] + ypp_ref[1]
    yn = ynp_ref[0] + ynp_ref[1]
    ap = jnp.maximum(disp * yp + b1p_ref[...], 0.0)
    an = jnp.maximum(disn * yn + b1n_ref[...], 0.0)
    h = ap - an
    hp2 = jnp.dot(h, w2p_ref[...], preferred_element_type=jnp.float32)
    hn2 = jnp.dot(h, w2n_ref[...], preferred_element_type=jnp.float32)
    gp2_ref[...] = hp2 * disp
    gn2_ref[...] = hn2 * disn


_dense2 = pl.pallas_call(
    _dense2_body,
    grid=(_GRID,),
    in_specs=[
        pl.BlockSpec((2, _BR, D), lambda i: (0, i, 0)),
        pl.BlockSpec((2, _BR, D), lambda i: (0, i, 0)),
        pl.BlockSpec((_BR, D), lambda i: (i, 0)),
        pl.BlockSpec((_BR, D), lambda i: (i, 0)),
        pl.BlockSpec((1, D), lambda i: (0, 0)),
        pl.BlockSpec((1, D), lambda i: (0, 0)),
        pl.BlockSpec((D, D), lambda i: (0, 0)),
        pl.BlockSpec((D, D), lambda i: (0, 0)),
    ],
    out_specs=[pl.BlockSpec((_BR, D), lambda i: (i, 0))] * 2,
    out_shape=[jax.ShapeDtypeStruct((NP, D), jnp.float32)] * 2,
    compiler_params=_tc_params,
)


def _dense3_body(ypp_ref, ynp_ref, dp_ref, dn_ref,
                 b2p_ref, b2n_ref, out_ref):
    disp = dp_ref[...]
    disn = dn_ref[...]
    yp = ypp_ref[0] + ypp_ref[1]
    yn = ynp_ref[0] + ynp_ref[1]
    op = jnp.maximum(disp * yp + b2p_ref[...], 0.0)
    on = jnp.maximum(disn * yn + b2n_ref[...], 0.0)
    o = op - on
    m = jnp.max(o, axis=1, keepdims=True)
    lse = jnp.log(jnp.sum(jnp.exp(o - m), axis=1, keepdims=True)) + m
    out_ref[...] = o - lse


_dense3 = pl.pallas_call(
    _dense3_body,
    grid=(_GRID,),
    in_specs=[
        pl.BlockSpec((2, _BR, D), lambda i: (0, i, 0)),
        pl.BlockSpec((2, _BR, D), lambda i: (0, i, 0)),
        pl.BlockSpec((_BR, D), lambda i: (i, 0)),
        pl.BlockSpec((_BR, D), lambda i: (i, 0)),
        pl.BlockSpec((1, D), lambda i: (0, 0)),
        pl.BlockSpec((1, D), lambda i: (0, 0)),
    ],
    out_specs=pl.BlockSpec((_BR, D), lambda i: (i, 0)),
    out_shape=jax.ShapeDtypeStruct((NP, D), jnp.float32),
    compiler_params=_tc_params,
)


def _pad_edges(v, loop):
    v = v.astype(jnp.int32)
    return jnp.concatenate(
        [v, loop, jnp.full((EP - E - N_NODES,), NP - 1, jnp.int32)]
    ).reshape(NW, NCH, CH)


def kernel(x, edge_index_pos, edge_index_neg,
           W1p, b1p, W1n, b1n, W2p, b2p, W2n, b2n):
    loop = jnp.arange(N_NODES, dtype=jnp.int32)
    sp_r = _pad_edges(edge_index_pos[0], loop)
    dp_r = _pad_edges(edge_index_pos[1], loop)
    sn_r = _pad_edges(edge_index_neg[0], loop)
    dn_r = _pad_edges(edge_index_neg[1], loop)
    xp = jnp.pad(x, ((0, NP - N_NODES), (0, 0)))
    zeros_tbl = jnp.zeros((NP, D), jnp.float32)
    ones_tbl = jnp.ones((CH, D), jnp.float32)

    deg = _deg_kernel(dp_r, dn_r, zeros_tbl, ones_tbl)
    gp, gn, disp, disn = _dense1(xp, W1p, W1n, deg)
    ypp, ynp = _conv_kernel(gp, gn, zeros_tbl, sp_r, dp_r, sn_r, dn_r)
    gp2, gn2 = _dense2(ypp, ynp, disp, disn,
                       b1p.reshape(1, D), b1n.reshape(1, D), W2p, W2n)
    ypp2, ynp2 = _conv_kernel(gp2, gn2, zeros_tbl, sp_r, dp_r, sn_r, dn_r)
    o = _dense3(ypp2, ynp2, disp, disn,
                b2p.reshape(1, D), b2n.reshape(1, D))
    return o[:N_NODES]


# speedup vs baseline: 58.7688x; 1.1321x over previous
import functools

import jax
import jax.numpy as jnp
from jax import lax
from jax.experimental import pallas as pl
from jax.experimental.pallas import tpu as pltpu
from jax.experimental.pallas import tpu_sc as plsc

N_NODES = 10000
NP = 10240
D = 16
F = 128
E = 320000
NW = 32
CH = 128
NCH = 81
KG = 9
EW = NCH * CH
EP = NW * EW

_mesh = plsc.VectorSubcoreMesh(core_axis_name="c", subcore_axis_name="s")
_sc_params = pltpu.CompilerParams(use_tc_tiling_on_sc=False,
                                  skip_device_barrier=True)
_tc_params = pltpu.CompilerParams(skip_device_barrier=True)


@functools.partial(
    pl.kernel,
    out_type=jax.ShapeDtypeStruct((2, 2, NP, D), jnp.float32),
    mesh=_mesh,
    scratch_types=[
        pltpu.VMEM((NCH, CH), jnp.int32),
        pltpu.VMEM((NCH, CH), jnp.int32),
        pltpu.VMEM((CH, D), jnp.float32),
        pltpu.SemaphoreType.DMA,
        pltpu.VMEM_SHARED((NP, D), jnp.float32),
        pltpu.VMEM_SHARED((NP, D), jnp.float32),
    ],
    compiler_params=_sc_params,
)
def _deg_kernel(dp_hbm, dn_hbm, z_hbm, ones_hbm, out_hbm,
                dpv, dnv, ones_v, sem_s, accp, accn):
    cid = lax.axis_index("c")
    sid = lax.axis_index("s")
    wid = cid * 16 + sid
    st = NP // 16
    pltpu.sync_copy(z_hbm.at[pl.ds(sid * st, st)], accp.at[pl.ds(sid * st, st)])
    pltpu.sync_copy(z_hbm.at[pl.ds(sid * st, st)], accn.at[pl.ds(sid * st, st)])
    pltpu.sync_copy(ones_hbm, ones_v)
    pltpu.sync_copy(dp_hbm.at[wid], dpv)
    pltpu.sync_copy(dn_hbm.at[wid], dnv)
    plsc.subcore_barrier()

    def one_sign(dv, acc):
        def fire(j, carry):
            pltpu.async_copy(ones_v, acc.at[dv.at[j]], sem_s, add=True)
            return carry
        lax.fori_loop(0, NCH, fire, 0)

        def drain(j, carry):
            pltpu.make_async_copy(ones_v, acc.at[dv.at[0]], sem_s).wait()
            return carry
        lax.fori_loop(0, NCH, drain, 0)

    one_sign(dpv, accp)
    one_sign(dnv, accn)
    plsc.subcore_barrier()
    pltpu.sync_copy(accp.at[pl.ds(sid * st, st)],
                    out_hbm.at[cid, 0, pl.ds(sid * st, st)])
    pltpu.sync_copy(accn.at[pl.ds(sid * st, st)],
                    out_hbm.at[cid, 1, pl.ds(sid * st, st)])


@functools.partial(
    pl.kernel,
    out_type=[jax.ShapeDtypeStruct((2, NP, D), jnp.float32),
              jax.ShapeDtypeStruct((2, NP, D), jnp.float32)],
    mesh=_mesh,
    scratch_types=[
        pltpu.VMEM((NCH, CH), jnp.int32),
        pltpu.VMEM((NCH, CH), jnp.int32),
        pltpu.VMEM((NCH, CH), jnp.int32),
        pltpu.VMEM((NCH, CH), jnp.int32),
        pltpu.VMEM((2, KG, CH, D), jnp.float32),
        pltpu.SemaphoreType.DMA,
        pltpu.SemaphoreType.DMA,
        pltpu.VMEM_SHARED((NP, D), jnp.float32),
        pltpu.VMEM_SHARED((NP, D), jnp.float32),
        pltpu.VMEM_SHARED((NP, D), jnp.float32),
        pltpu.VMEM_SHARED((NP, D), jnp.float32),
    ],
    compiler_params=_sc_params,
)
def _conv_kernel(gp_hbm, gn_hbm, z_hbm, sp_hbm, dp_hbm, sn_hbm, dn_hbm,
                 yp_hbm, yn_hbm, spv, dpv, snv, dnv, rows, sem_g, sem_s,
                 accp, accn, gsp, gsn):
    cid = lax.axis_index("c")
    sid = lax.axis_index("s")
    wid = cid * 16 + sid
    st = NP // 16
    pltpu.sync_copy(z_hbm.at[pl.ds(sid * st, st)], accp.at[pl.ds(sid * st, st)])
    pltpu.sync_copy(z_hbm.at[pl.ds(sid * st, st)], accn.at[pl.ds(sid * st, st)])
    pltpu.sync_copy(gp_hbm.at[pl.ds(sid * st, st)], gsp.at[pl.ds(sid * st, st)])
    pltpu.sync_copy(gn_hbm.at[pl.ds(sid * st, st)], gsn.at[pl.ds(sid * st, st)])
    pltpu.sync_copy(sp_hbm.at[wid], spv)
    pltpu.sync_copy(dp_hbm.at[wid], dpv)
    pltpu.sync_copy(sn_hbm.at[wid], snv)
    pltpu.sync_copy(dn_hbm.at[wid], dnv)
    plsc.subcore_barrier()

    NG = NCH // KG

    def one_sign(g_hbm, sv, dv, acc):
        for k in range(KG):
            pltpu.async_copy(g_hbm.at[sv.at[k]], rows.at[0, k], sem_g)

        def group(t, carry):
            par = lax.rem(t, 2)
            nxt = 1 - par
            base = t * KG

            @pl.when(t + 1 < NG)
            def _fire_next():
                @pl.when(t >= 1)
                def _drain_prev_scatters():
                    for k in range(KG):
                        pltpu.make_async_copy(
                            rows.at[nxt, k],
                            acc.at[dv.at[base - KG + k]], sem_s).wait()
                for k in range(KG):
                    pltpu.async_copy(g_hbm.at[sv.at[base + KG + k]],
                                     rows.at[nxt, k], sem_g)

            for k in range(KG):
                pltpu.make_async_copy(g_hbm.at[sv.at[base + k]],
                                      rows.at[par, k], sem_g).wait()
                pltpu.async_copy(rows.at[par, k], acc.at[dv.at[base + k]],
                                 sem_s, add=True)
            return carry

        lax.fori_loop(0, NG, group, 0)
        for k in range(2 * KG):
            pltpu.make_async_copy(rows.at[0, 0], acc.at[dv.at[0]],
                                  sem_s).wait()

    one_sign(gsp, spv, dpv, accp)
    one_sign(gsn, snv, dnv, accn)
    plsc.subcore_barrier()
    pltpu.sync_copy(accp.at[pl.ds(sid * st, st)],
                    yp_hbm.at[cid, pl.ds(sid * st, st)])
    pltpu.sync_copy(accn.at[pl.ds(sid * st, st)],
                    yn_hbm.at[cid, pl.ds(sid * st, st)])


_GRID = 8
_BR = NP // _GRID


def _dis(deg):
    return jnp.where(deg > 0.0, lax.rsqrt(deg), 0.0)


def _dense1_body(x_ref, w1p_ref, w1n_ref, deg_ref,
                 gp_ref, gn_ref, dp_ref, dn_ref):
    deg = deg_ref[...]
    disp = _dis(deg[0, 0] + deg[1, 0])
    disn = _dis(deg[0, 1] + deg[1, 1])
    hp = jnp.dot(x_ref[...], w1p_ref[...], preferred_element_type=jnp.float32)
    hn = jnp.dot(x_ref[...], w1n_ref[...], preferred_element_type=jnp.float32)
    gp_ref[...] = hp * disp
    gn_ref[...] = hn * disn
    dp_ref[...] = disp
    dn_ref[...] = disn


_dense1 = pl.pallas_call(
    _dense1_body,
    grid=(_GRID,),
    in_specs=[
        pl.BlockSpec((_BR, F), lambda i: (i, 0)),
        pl.BlockSpec((F, D), lambda i: (0, 0)),
        pl.BlockSpec((F, D), lambda i: (0, 0)),
        pl.BlockSpec((2, 2, _BR, D), lambda i: (0, 0, i, 0)),
    ],
    out_specs=[pl.BlockSpec((_BR, D), lambda i: (i, 0))] * 4,
    out_shape=[jax.ShapeDtypeStruct((NP, D), jnp.float32)] * 4,
    compiler_params=_tc_params,
)


def _dense2_body(ypp_ref, ynp_ref, dp_ref, dn_ref,
                 b1p_ref, b1n_ref, w2p_ref, w2n_ref,
                 gp2_ref, gn2_ref):
    disp = dp_ref[...]
    disn = dn_ref[...]
    yp = ypp_ref[0] + ypp_ref[1]
    yn = ynp_ref[0] + ynp_ref[1]
    ap = jnp.maximum(disp * yp + b1p_ref[...], 0.0)
    an = jnp.maximum(disn * yn + b1n_ref[...], 0.0)
    h = ap - an
    hp2 = jnp.dot(h, w2p_ref[...], preferred_element_type=jnp.float32)
    hn2 = jnp.dot(h, w2n_ref[...], preferred_element_type=jnp.float32)
    gp2_ref[...] = hp2 * disp
    gn2_ref[...] = hn2 * disn


_dense2 = pl.pallas_call(
    _dense2_body,
    grid=(_GRID,),
    in_specs=[
        pl.BlockSpec((2, _BR, D), lambda i: (0, i, 0)),
        pl.BlockSpec((2, _BR, D), lambda i: (0, i, 0)),
        pl.BlockSpec((_BR, D), lambda i: (i, 0)),
        pl.BlockSpec((_BR, D), lambda i: (i, 0)),
        pl.BlockSpec((1, D), lambda i: (0, 0)),
        pl.BlockSpec((1, D), lambda i: (0, 0)),
        pl.BlockSpec((D, D), lambda i: (0, 0)),
        pl.BlockSpec((D, D), lambda i: (0, 0)),
    ],
    out_specs=[pl.BlockSpec((_BR, D), lambda i: (i, 0))] * 2,
    out_shape=[jax.ShapeDtypeStruct((NP, D), jnp.float32)] * 2,
    compiler_params=_tc_params,
)


def _dense3_body(ypp_ref, ynp_ref, dp_ref, dn_ref,
                 b2p_ref, b2n_ref, out_ref):
    disp = dp_ref[...]
    disn = dn_ref[...]
    yp = ypp_ref[0] + ypp_ref[1]
    yn = ynp_ref[0] + ynp_ref[1]
    op = jnp.maximum(disp * yp + b2p_ref[...], 0.0)
    on = jnp.maximum(disn * yn + b2n_ref[...], 0.0)
    o = op - on
    m = jnp.max(o, axis=1, keepdims=True)
    lse = jnp.log(jnp.sum(jnp.exp(o - m), axis=1, keepdims=True)) + m
    out_ref[...] = o - lse


_dense3 = pl.pallas_call(
    _dense3_body,
    grid=(_GRID,),
    in_specs=[
        pl.BlockSpec((2, _BR, D), lambda i: (0, i, 0)),
        pl.BlockSpec((2, _BR, D), lambda i: (0, i, 0)),
        pl.BlockSpec((_BR, D), lambda i: (i, 0)),
        pl.BlockSpec((_BR, D), lambda i: (i, 0)),
        pl.BlockSpec((1, D), lambda i: (0, 0)),
        pl.BlockSpec((1, D), lambda i: (0, 0)),
    ],
    out_specs=pl.BlockSpec((_BR, D), lambda i: (i, 0)),
    out_shape=jax.ShapeDtypeStruct((NP, D), jnp.float32),
    compiler_params=_tc_params,
)


def _pad_edges(v, loop):
    v = v.astype(jnp.int32)
    return jnp.concatenate(
        [v, loop, jnp.full((EP - E - N_NODES,), NP - 1, jnp.int32)]
    ).reshape(NW, NCH, CH)


def kernel(x, edge_index_pos, edge_index_neg,
           W1p, b1p, W1n, b1n, W2p, b2p, W2n, b2n):
    loop = jnp.arange(N_NODES, dtype=jnp.int32)
    sp_r = _pad_edges(edge_index_pos[0], loop)
    dp_r = _pad_edges(edge_index_pos[1], loop)
    sn_r = _pad_edges(edge_index_neg[0], loop)
    dn_r = _pad_edges(edge_index_neg[1], loop)
    xp = jnp.pad(x, ((0, NP - N_NODES), (0, 0)))
    zeros_tbl = jnp.zeros((NP, D), jnp.float32)
    ones_tbl = jnp.ones((CH, D), jnp.float32)

    deg = _deg_kernel(dp_r, dn_r, zeros_tbl, ones_tbl)
    gp, gn, disp, disn = _dense1(xp, W1p, W1n, deg)
    ypp, ynp = _conv_kernel(gp, gn, zeros_tbl, sp_r, dp_r, sn_r, dn_r)
    gp2, gn2 = _dense2(ypp, ynp, disp, disn,
                       b1p.reshape(1, D), b1n.reshape(1, D), W2p, W2n)
    ypp2, ynp2 = _conv_kernel(gp2, gn2, zeros_tbl, sp_r, dp_r, sn_r, dn_r)
    o = _dense3(ypp2, ynp2, disp, disn,
                b2p.reshape(1, D), b2n.reshape(1, D))
    return o[:N_NODES]


# speedup vs baseline: 61.3808x; 1.0444x over previous
import functools

import jax
import jax.numpy as jnp
from jax import lax
from jax.experimental import pallas as pl
from jax.experimental.pallas import tpu as pltpu
from jax.experimental.pallas import tpu_sc as plsc

N_NODES = 10000
NP = 10240
D = 16
F = 128
E = 320000
NW = 32
CH = 128
NCH = 81
KG = 9
EW = NCH * CH
EP = NW * EW

_mesh = plsc.VectorSubcoreMesh(core_axis_name="c", subcore_axis_name="s")
_sc_params = pltpu.CompilerParams(use_tc_tiling_on_sc=False,
                                  skip_device_barrier=True)
_tc_params = pltpu.CompilerParams(skip_device_barrier=True)


@functools.partial(
    pl.kernel,
    out_type=jax.ShapeDtypeStruct((2, 2, N_NODES, D), jnp.float32),
    mesh=_mesh,
    scratch_types=[
        pltpu.VMEM((NCH, CH), jnp.int32),
        pltpu.VMEM((NCH, CH), jnp.int32),
        pltpu.VMEM((CH, D), jnp.float32),
        pltpu.SemaphoreType.DMA,
        pltpu.VMEM_SHARED((NP, D), jnp.float32),
        pltpu.VMEM_SHARED((NP, D), jnp.float32),
    ],
    compiler_params=_sc_params,
)
def _deg_kernel(dp_hbm, dn_hbm, z_hbm, ones_hbm, out_hbm,
                dpv, dnv, ones_v, sem_s, accp, accn):
    cid = lax.axis_index("c")
    sid = lax.axis_index("s")
    wid = cid * 16 + sid
    st = NP // 16
    stg = N_NODES // 16
    pltpu.sync_copy(z_hbm.at[pl.ds(sid * st, st)], accp.at[pl.ds(sid * st, st)])
    pltpu.sync_copy(z_hbm.at[pl.ds(sid * st, st)], accn.at[pl.ds(sid * st, st)])
    pltpu.sync_copy(ones_hbm, ones_v)
    pltpu.sync_copy(dp_hbm.at[wid], dpv)
    pltpu.sync_copy(dn_hbm.at[wid], dnv)
    plsc.subcore_barrier()

    def one_sign(dv, acc):
        def fire(j, carry):
            pltpu.async_copy(ones_v, acc.at[dv.at[j]], sem_s, add=True)
            return carry
        lax.fori_loop(0, NCH, fire, 0)

        def drain(j, carry):
            pltpu.make_async_copy(ones_v, acc.at[dv.at[0]], sem_s).wait()
            return carry
        lax.fori_loop(0, NCH, drain, 0)

    one_sign(dpv, accp)
    one_sign(dnv, accn)
    plsc.subcore_barrier()
    pltpu.sync_copy(accp.at[pl.ds(sid * stg, stg)],
                    out_hbm.at[cid, 0, pl.ds(sid * stg, stg)])
    pltpu.sync_copy(accn.at[pl.ds(sid * stg, stg)],
                    out_hbm.at[cid, 1, pl.ds(sid * stg, stg)])


@functools.partial(
    pl.kernel,
    out_type=[jax.ShapeDtypeStruct((2, N_NODES, D), jnp.float32),
              jax.ShapeDtypeStruct((2, N_NODES, D), jnp.float32)],
    mesh=_mesh,
    scratch_types=[
        pltpu.VMEM((NCH, CH), jnp.int32),
        pltpu.VMEM((NCH, CH), jnp.int32),
        pltpu.VMEM((NCH, CH), jnp.int32),
        pltpu.VMEM((NCH, CH), jnp.int32),
        pltpu.VMEM((2, KG, CH, D), jnp.float32),
        pltpu.SemaphoreType.DMA,
        pltpu.SemaphoreType.DMA,
        pltpu.VMEM_SHARED((NP, D), jnp.float32),
        pltpu.VMEM_SHARED((NP, D), jnp.float32),
        pltpu.VMEM_SHARED((N_NODES, D), jnp.float32),
        pltpu.VMEM_SHARED((N_NODES, D), jnp.float32),
    ],
    compiler_params=_sc_params,
)
def _conv_kernel(gp_hbm, gn_hbm, z_hbm, sp_hbm, dp_hbm, sn_hbm, dn_hbm,
                 yp_hbm, yn_hbm, spv, dpv, snv, dnv, rows, sem_g, sem_s,
                 accp, accn, gsp, gsn):
    cid = lax.axis_index("c")
    sid = lax.axis_index("s")
    wid = cid * 16 + sid
    st = NP // 16
    stg = N_NODES // 16
    pltpu.sync_copy(z_hbm.at[pl.ds(sid * st, st)], accp.at[pl.ds(sid * st, st)])
    pltpu.sync_copy(z_hbm.at[pl.ds(sid * st, st)], accn.at[pl.ds(sid * st, st)])
    pltpu.sync_copy(gp_hbm.at[pl.ds(sid * stg, stg)], gsp.at[pl.ds(sid * stg, stg)])
    pltpu.sync_copy(gn_hbm.at[pl.ds(sid * stg, stg)], gsn.at[pl.ds(sid * stg, stg)])
    pltpu.sync_copy(sp_hbm.at[wid], spv)
    pltpu.sync_copy(dp_hbm.at[wid], dpv)
    pltpu.sync_copy(sn_hbm.at[wid], snv)
    pltpu.sync_copy(dn_hbm.at[wid], dnv)
    plsc.subcore_barrier()

    NG = NCH // KG

    def one_sign(g_hbm, sv, dv, acc):
        for k in range(KG):
            pltpu.async_copy(g_hbm.at[sv.at[k]], rows.at[0, k], sem_g)

        def group(t, carry):
            par = lax.rem(t, 2)
            nxt = 1 - par
            base = t * KG

            @pl.when(t + 1 < NG)
            def _fire_next():
                @pl.when(t >= 1)
                def _drain_prev_scatters():
                    for k in range(KG):
                        pltpu.make_async_copy(
                            rows.at[nxt, k],
                            acc.at[dv.at[base - KG + k]], sem_s).wait()
                for k in range(KG):
                    pltpu.async_copy(g_hbm.at[sv.at[base + KG + k]],
                                     rows.at[nxt, k], sem_g)

            for k in range(KG):
                pltpu.make_async_copy(g_hbm.at[sv.at[base + k]],
                                      rows.at[par, k], sem_g).wait()
                pltpu.async_copy(rows.at[par, k], acc.at[dv.at[base + k]],
                                 sem_s, add=True)
            return carry

        lax.fori_loop(0, NG, group, 0)
        for k in range(2 * KG):
            pltpu.make_async_copy(rows.at[0, 0], acc.at[dv.at[0]],
                                  sem_s).wait()

    one_sign(gsp, spv, dpv, accp)
    one_sign(gsn, snv, dnv, accn)
    plsc.subcore_barrier()
    pltpu.sync_copy(accp.at[pl.ds(sid * stg, stg)],
                    yp_hbm.at[cid, pl.ds(sid * stg, stg)])
    pltpu.sync_copy(accn.at[pl.ds(sid * stg, stg)],
                    yn_hbm.at[cid, pl.ds(sid * stg, stg)])


_GRID = 5
_BR = N_NODES // _GRID


def _dis(deg):
    return jnp.where(deg > 0.0, lax.rsqrt(deg), 0.0)


def _dense1_body(x_ref, w1p_ref, w1n_ref, deg_ref,
                 gp_ref, gn_ref, dp_ref, dn_ref):
    deg = deg_ref[...]
    disp = _dis(deg[0, 0] + deg[1, 0])
    disn = _dis(deg[0, 1] + deg[1, 1])
    hp = jnp.dot(x_ref[...], w1p_ref[...], preferred_element_type=jnp.float32)
    hn = jnp.dot(x_ref[...], w1n_ref[...], preferred_element_type=jnp.float32)
    gp_ref[...] = hp * disp
    gn_ref[...] = hn * disn
    dp_ref[...] = disp
    dn_ref[...] = disn


_dense1 = pl.pallas_call(
    _dense1_body,
    grid=(_GRID,),
    in_specs=[
        pl.BlockSpec((_BR, F), lambda i: (i, 0)),
        pl.BlockSpec((F, D), lambda i: (0, 0)),
        pl.BlockSpec((F, D), lambda i: (0, 0)),
        pl.BlockSpec((2, 2, _BR, D), lambda i: (0, 0, i, 0)),
    ],
    out_specs=[pl.BlockSpec((_BR, D), lambda i: (i, 0))] * 4,
    out_shape=[jax.ShapeDtypeStruct((N_NODES, D), jnp.float32)] * 4,
    compiler_params=_tc_params,
)


def _dense2_body(ypp_ref, ynp_ref, dp_ref, dn_ref,
                 b1p_ref, b1n_ref, w2p_ref, w2n_ref,
                 gp2_ref, gn2_ref):
    disp = dp_ref[...]
    disn = dn_ref[...]
    yp = ypp_ref[0] + ypp_ref[1]
    yn = ynp_ref[0] + ynp_ref[1]
    ap = jnp.maximum(disp * yp + b1p_ref[...], 0.0)
    an = jnp.maximum(disn * yn + b1n_ref[...], 0.0)
    h = ap - an
    hp2 = jnp.dot(h, w2p_ref[...], preferred_element_type=jnp.float32)
    hn2 = jnp.dot(h, w2n_ref[...], preferred_element_type=jnp.float32)
    gp2_ref[...] = hp2 * disp
    gn2_ref[...] = hn2 * disn


_dense2 = pl.pallas_call(
    _dense2_body,
    grid=(_GRID,),
    in_specs=[
        pl.BlockSpec((2, _BR, D), lambda i: (0, i, 0)),
        pl.BlockSpec((2, _BR, D), lambda i: (0, i, 0)),
        pl.BlockSpec((_BR, D), lambda i: (i, 0)),
        pl.BlockSpec((_BR, D), lambda i: (i, 0)),
        pl.BlockSpec((1, D), lambda i: (0, 0)),
        pl.BlockSpec((1, D), lambda i: (0, 0)),
        pl.BlockSpec((D, D), lambda i: (0, 0)),
        pl.BlockSpec((D, D), lambda i: (0, 0)),
    ],
    out_specs=[pl.BlockSpec((_BR, D), lambda i: (i, 0))] * 2,
    out_shape=[jax.ShapeDtypeStruct((N_NODES, D), jnp.float32)] * 2,
    compiler_params=_tc_params,
)


def _dense3_body(ypp_ref, ynp_ref, dp_ref, dn_ref,
                 b2p_ref, b2n_ref, out_ref):
    disp = dp_ref[...]
    disn = dn_ref[...]
    yp = ypp_ref[0] + ypp_ref[1]
    yn = ynp_ref[0] + ynp_ref[1]
    op = jnp.maximum(disp * yp + b2p_ref[...], 0.0)
    on = jnp.maximum(disn * yn + b2n_ref[...], 0.0)
    o = op - on
    m = jnp.max(o, axis=1, keepdims=True)
    lse = jnp.log(jnp.sum(jnp.exp(o - m), axis=1, keepdims=True)) + m
    out_ref[...] = o - lse


_dense3 = pl.pallas_call(
    _dense3_body,
    grid=(_GRID,),
    in_specs=[
        pl.BlockSpec((2, _BR, D), lambda i: (0, i, 0)),
        pl.BlockSpec((2, _BR, D), lambda i: (0, i, 0)),
        pl.BlockSpec((_BR, D), lambda i: (i, 0)),
        pl.BlockSpec((_BR, D), lambda i: (i, 0)),
        pl.BlockSpec((1, D), lambda i: (0, 0)),
        pl.BlockSpec((1, D), lambda i: (0, 0)),
    ],
    out_specs=pl.BlockSpec((_BR, D), lambda i: (i, 0)),
    out_shape=jax.ShapeDtypeStruct((N_NODES, D), jnp.float32),
    compiler_params=_tc_params,
)


def _pad_edges(v, loop, fill):
    v = v.astype(jnp.int32)
    return jnp.concatenate(
        [v, loop, jnp.full((EP - E - N_NODES,), fill, jnp.int32)]
    ).reshape(NW, NCH, CH)


def kernel(x, edge_index_pos, edge_index_neg,
           W1p, b1p, W1n, b1n, W2p, b2p, W2n, b2n):
    loop = jnp.arange(N_NODES, dtype=jnp.int32)
    sp_r = _pad_edges(edge_index_pos[0], loop, 0)
    dp_r = _pad_edges(edge_index_pos[1], loop, NP - 1)
    sn_r = _pad_edges(edge_index_neg[0], loop, 0)
    dn_r = _pad_edges(edge_index_neg[1], loop, NP - 1)
    zeros_tbl = jnp.zeros((NP, D), jnp.float32)
    ones_tbl = jnp.ones((CH, D), jnp.float32)

    deg = _deg_kernel(dp_r, dn_r, zeros_tbl, ones_tbl)
    gp, gn, disp, disn = _dense1(x, W1p, W1n, deg)
    ypp, ynp = _conv_kernel(gp, gn, zeros_tbl, sp_r, dp_r, sn_r, dn_r)
    gp2, gn2 = _dense2(ypp, ynp, disp, disn,
                       b1p.reshape(1, D), b1n.reshape(1, D), W2p, W2n)
    ypp2, ynp2 = _conv_kernel(gp2, gn2, zeros_tbl, sp_r, dp_r, sn_r, dn_r)
    o = _dense3(ypp2, ynp2, disp, disn,
                b2p.reshape(1, D), b2n.reshape(1, D))
    return o


# speedup vs baseline: 68.8411x; 1.1215x over previous
import functools

import jax
import jax.numpy as jnp
from jax import lax
from jax.experimental import pallas as pl
from jax.experimental.pallas import tpu as pltpu
from jax.experimental.pallas import tpu_sc as plsc

N_NODES = 10000
NP = 10240
D = 16
F = 128
E = 320000
NW = 32
CH = 128
NCH = 81
KG = 9
EW = NCH * CH
EP = NW * EW
EC = E // CH
LC = (EP - E) // CH
B30 = EC - 30 * NCH

_mesh = plsc.VectorSubcoreMesh(core_axis_name="c", subcore_axis_name="s")
_sc_params = pltpu.CompilerParams(use_tc_tiling_on_sc=False,
                                  skip_device_barrier=True)
_tc_params = pltpu.CompilerParams(skip_device_barrier=True)


def _load_idx(tbl_hbm, loops_hbm, row, buf, wid):
    @pl.when(wid <= 29)
    def _all_edges():
        pltpu.sync_copy(tbl_hbm.at[row, pl.ds(wid * NCH, NCH)], buf)

    @pl.when(wid == 30)
    def _boundary():
        pltpu.sync_copy(tbl_hbm.at[row, pl.ds(30 * NCH, B30)],
                        buf.at[pl.ds(0, B30)])
        pltpu.sync_copy(loops_hbm.at[row, pl.ds(0, NCH - B30)],
                        buf.at[pl.ds(B30, NCH - B30)])

    @pl.when(wid == 31)
    def _all_loops():
        pltpu.sync_copy(loops_hbm.at[row, pl.ds(NCH - B30, NCH)], buf)


@functools.partial(
    pl.kernel,
    out_type=jax.ShapeDtypeStruct((2, 2, N_NODES, D), jnp.float32),
    mesh=_mesh,
    scratch_types=[
        pltpu.VMEM((NCH, CH), jnp.int32),
        pltpu.VMEM((NCH, CH), jnp.int32),
        pltpu.VMEM((CH, D), jnp.float32),
        pltpu.SemaphoreType.DMA,
        pltpu.VMEM_SHARED((NP, D), jnp.float32),
        pltpu.VMEM_SHARED((NP, D), jnp.float32),
    ],
    compiler_params=_sc_params,
)
def _deg_kernel(eip_hbm, ein_hbm, loops_hbm, z_hbm, ones_hbm, out_hbm,
                dpv, dnv, ones_v, sem_s, accp, accn):
    cid = lax.axis_index("c")
    sid = lax.axis_index("s")
    wid = cid * 16 + sid
    st = NP // 16
    stg = N_NODES // 16
    pltpu.sync_copy(z_hbm.at[pl.ds(sid * st, st)], accp.at[pl.ds(sid * st, st)])
    pltpu.sync_copy(z_hbm.at[pl.ds(sid * st, st)], accn.at[pl.ds(sid * st, st)])
    pltpu.sync_copy(ones_hbm, ones_v)
    _load_idx(eip_hbm, loops_hbm, 1, dpv, wid)
    _load_idx(ein_hbm, loops_hbm, 1, dnv, wid)
    plsc.subcore_barrier()

    def one_sign(dv, acc):
        def fire(j, carry):
            pltpu.async_copy(ones_v, acc.at[dv.at[j]], sem_s, add=True)
            return carry
        lax.fori_loop(0, NCH, fire, 0)

        def drain(j, carry):
            pltpu.make_async_copy(ones_v, acc.at[dv.at[0]], sem_s).wait()
            return carry
        lax.fori_loop(0, NCH, drain, 0)

    one_sign(dpv, accp)
    one_sign(dnv, accn)
    plsc.subcore_barrier()
    pltpu.sync_copy(accp.at[pl.ds(sid * stg, stg)],
                    out_hbm.at[cid, 0, pl.ds(sid * stg, stg)])
    pltpu.sync_copy(accn.at[pl.ds(sid * stg, stg)],
                    out_hbm.at[cid, 1, pl.ds(sid * stg, stg)])


@functools.partial(
    pl.kernel,
    out_type=[jax.ShapeDtypeStruct((2, N_NODES, D), jnp.float32),
              jax.ShapeDtypeStruct((2, N_NODES, D), jnp.float32)],
    mesh=_mesh,
    scratch_types=[
        pltpu.VMEM((NCH, CH), jnp.int32),
        pltpu.VMEM((NCH, CH), jnp.int32),
        pltpu.VMEM((NCH, CH), jnp.int32),
        pltpu.VMEM((NCH, CH), jnp.int32),
        pltpu.VMEM((2, KG, CH, D), jnp.float32),
        pltpu.SemaphoreType.DMA,
        pltpu.SemaphoreType.DMA,
        pltpu.VMEM_SHARED((NP, D), jnp.float32),
        pltpu.VMEM_SHARED((NP, D), jnp.float32),
        pltpu.VMEM_SHARED((N_NODES, D), jnp.float32),
        pltpu.VMEM_SHARED((N_NODES, D), jnp.float32),
    ],
    compiler_params=_sc_params,
)
def _conv_kernel(gp_hbm, gn_hbm, z_hbm, eip_hbm, ein_hbm, loops_hbm,
                 yp_hbm, yn_hbm, spv, dpv, snv, dnv, rows, sem_g, sem_s,
                 accp, accn, gsp, gsn):
    cid = lax.axis_index("c")
    sid = lax.axis_index("s")
    wid = cid * 16 + sid
    st = NP // 16
    stg = N_NODES // 16
    pltpu.sync_copy(z_hbm.at[pl.ds(sid * st, st)], accp.at[pl.ds(sid * st, st)])
    pltpu.sync_copy(z_hbm.at[pl.ds(sid * st, st)], accn.at[pl.ds(sid * st, st)])
    pltpu.sync_copy(gp_hbm.at[pl.ds(sid * stg, stg)], gsp.at[pl.ds(sid * stg, stg)])
    pltpu.sync_copy(gn_hbm.at[pl.ds(sid * stg, stg)], gsn.at[pl.ds(sid * stg, stg)])
    _load_idx(eip_hbm, loops_hbm, 0, spv, wid)
    _load_idx(eip_hbm, loops_hbm, 1, dpv, wid)
    _load_idx(ein_hbm, loops_hbm, 0, snv, wid)
    _load_idx(ein_hbm, loops_hbm, 1, dnv, wid)
    plsc.subcore_barrier()

    NG = NCH // KG

    def one_sign(g_hbm, sv, dv, acc):
        for k in range(KG):
            pltpu.async_copy(g_hbm.at[sv.at[k]], rows.at[0, k], sem_g)

        def group(t, carry):
            par = lax.rem(t, 2)
            nxt = 1 - par
            base = t * KG

            @pl.when(t + 1 < NG)
            def _fire_next():
                @pl.when(t >= 1)
                def _drain_prev_scatters():
                    for k in range(KG):
                        pltpu.make_async_copy(
                            rows.at[nxt, k],
                            acc.at[dv.at[base - KG + k]], sem_s).wait()
                for k in range(KG):
                    pltpu.async_copy(g_hbm.at[sv.at[base + KG + k]],
                                     rows.at[nxt, k], sem_g)

            for k in range(KG):
                pltpu.make_async_copy(g_hbm.at[sv.at[base + k]],
                                      rows.at[par, k], sem_g).wait()
                pltpu.async_copy(rows.at[par, k], acc.at[dv.at[base + k]],
                                 sem_s, add=True)
            return carry

        lax.fori_loop(0, NG, group, 0)
        for k in range(2 * KG):
            pltpu.make_async_copy(rows.at[0, 0], acc.at[dv.at[0]],
                                  sem_s).wait()

    one_sign(gsp, spv, dpv, accp)
    one_sign(gsn, snv, dnv, accn)
    plsc.subcore_barrier()
    pltpu.sync_copy(accp.at[pl.ds(sid * stg, stg)],
                    yp_hbm.at[cid, pl.ds(sid * stg, stg)])
    pltpu.sync_copy(accn.at[pl.ds(sid * stg, stg)],
                    yn_hbm.at[cid, pl.ds(sid * stg, stg)])


_GRID = 5
_BR = N_NODES // _GRID


def _dis(deg):
    return jnp.where(deg > 0.0, lax.rsqrt(deg), 0.0)


def _dense1_body(x_ref, w1p_ref, w1n_ref, deg_ref,
                 gp_ref, gn_ref, dp_ref, dn_ref):
    deg = deg_ref[...]
    disp = _dis(deg[0, 0] + deg[1, 0])
    disn = _dis(deg[0, 1] + deg[1, 1])
    hp = jnp.dot(x_ref[...], w1p_ref[...], preferred_element_type=jnp.float32)
    hn = jnp.dot(x_ref[...], w1n_ref[...], preferred_element_type=jnp.float32)
    gp_ref[...] = hp * disp
    gn_ref[...] = hn * disn
    dp_ref[...] = disp
    dn_ref[...] = disn


_dense1 = pl.pallas_call(
    _dense1_body,
    grid=(_GRID,),
    in_specs=[
        pl.BlockSpec((_BR, F), lambda i: (i, 0)),
        pl.BlockSpec((F, D), lambda i: (0, 0)),
        pl.BlockSpec((F, D), lambda i: (0, 0)),
        pl.BlockSpec((2, 2, _BR, D), lambda i: (0, 0, i, 0)),
    ],
    out_specs=[pl.BlockSpec((_BR, D), lambda i: (i, 0))] * 4,
    out_shape=[jax.ShapeDtypeStruct((N_NODES, D), jnp.float32)] * 4,
    compiler_params=_tc_params,
)


def _dense2_body(ypp_ref, ynp_ref, dp_ref, dn_ref,
                 b1p_ref, b1n_ref, w2p_ref, w2n_ref,
                 gp2_ref, gn2_ref):
    disp = dp_ref[...]
    disn = dn_ref[...]
    yp = ypp_ref[0] + ypp_ref[1]
    yn = ynp_ref[0] + ynp_ref[1]
    ap = jnp.maximum(disp * yp + b1p_ref[...], 0.0)
    an = jnp.maximum(disn * yn + b1n_ref[...], 0.0)
    h = ap - an
    hp2 = jnp.dot(h, w2p_ref[...], preferred_element_type=jnp.float32)
    hn2 = jnp.dot(h, w2n_ref[...], preferred_element_type=jnp.float32)
    gp2_ref[...] = hp2 * disp
    gn2_ref[...] = hn2 * disn


_dense2 = pl.pallas_call(
    _dense2_body,
    grid=(_GRID,),
    in_specs=[
        pl.BlockSpec((2, _BR, D), lambda i: (0, i, 0)),
        pl.BlockSpec((2, _BR, D), lambda i: (0, i, 0)),
        pl.BlockSpec((_BR, D), lambda i: (i, 0)),
        pl.BlockSpec((_BR, D), lambda i: (i, 0)),
        pl.BlockSpec((1, D), lambda i: (0, 0)),
        pl.BlockSpec((1, D), lambda i: (0, 0)),
        pl.BlockSpec((D, D), lambda i: (0, 0)),
        pl.BlockSpec((D, D), lambda i: (0, 0)),
    ],
    out_specs=[pl.BlockSpec((_BR, D), lambda i: (i, 0))] * 2,
    out_shape=[jax.ShapeDtypeStruct((N_NODES, D), jnp.float32)] * 2,
    compiler_params=_tc_params,
)


def _dense3_body(ypp_ref, ynp_ref, dp_ref, dn_ref,
                 b2p_ref, b2n_ref, out_ref):
    disp = dp_ref[...]
    disn = dn_ref[...]
    yp = ypp_ref[0] + ypp_ref[1]
    yn = ynp_ref[0] + ynp_ref[1]
    op = jnp.maximum(disp * yp + b2p_ref[...], 0.0)
    on = jnp.maximum(disn * yn + b2n_ref[...], 0.0)
    o = op - on
    m = jnp.max(o, axis=1, keepdims=True)
    lse = jnp.log(jnp.sum(jnp.exp(o - m), axis=1, keepdims=True)) + m
    out_ref[...] = o - lse


_dense3 = pl.pallas_call(
    _dense3_body,
    grid=(_GRID,),
    in_specs=[
        pl.BlockSpec((2, _BR, D), lambda i: (0, i, 0)),
        pl.BlockSpec((2, _BR, D), lambda i: (0, i, 0)),
        pl.BlockSpec((_BR, D), lambda i: (i, 0)),
        pl.BlockSpec((_BR, D), lambda i: (i, 0)),
        pl.BlockSpec((1, D), lambda i: (0, 0)),
        pl.BlockSpec((1, D), lambda i: (0, 0)),
    ],
    out_specs=pl.BlockSpec((_BR, D), lambda i: (i, 0)),
    out_shape=jax.ShapeDtypeStruct((N_NODES, D), jnp.float32),
    compiler_params=_tc_params,
)


def kernel(x, edge_index_pos, edge_index_neg,
           W1p, b1p, W1n, b1n, W2p, b2p, W2n, b2n):
    eip_r = edge_index_pos.astype(jnp.int32).reshape(2, EC, CH)
    ein_r = edge_index_neg.astype(jnp.int32).reshape(2, EC, CH)
    loop = jnp.arange(N_NODES, dtype=jnp.int32)
    loops_r = jnp.stack([
        jnp.concatenate([loop, jnp.zeros((LC * CH - N_NODES,), jnp.int32)]),
        jnp.concatenate([loop, jnp.full((LC * CH - N_NODES,), NP - 1,
                                        jnp.int32)]),
    ]).reshape(2, LC, CH)
    zeros_tbl = jnp.zeros((NP, D), jnp.float32)
    ones_tbl = jnp.ones((CH, D), jnp.float32)

    deg = _deg_kernel(eip_r, ein_r, loops_r, zeros_tbl, ones_tbl)
    gp, gn, disp, disn = _dense1(x, W1p, W1n, deg)
    ypp, ynp = _conv_kernel(gp, gn, zeros_tbl, eip_r, ein_r, loops_r)
    gp2, gn2 = _dense2(ypp, ynp, disp, disn,
                       b1p.reshape(1, D), b1n.reshape(1, D), W2p, W2n)
    ypp2, ynp2 = _conv_kernel(gp2, gn2, zeros_tbl, eip_r, ein_r, loops_r)
    o = _dense3(ypp2, ynp2, disp, disn,
                b2p.reshape(1, D), b2n.reshape(1, D))
    return o
